# Initial kernel scaffold; baseline (speedup 1.0000x reference)
import jax
import jax.numpy as jnp
from jax.experimental import pallas as pl


def kernel(x, edge_index, node_label, node_index, W1, att1, b1, W2, att2, b2, W3, att3, b3, outW, outb):
    raise NotImplementedError("write your pallas kernel here")


# speedup vs baseline: 10.9716x; 10.9716x over previous
import functools

import jax
import jax.numpy as jnp
from jax import lax
from jax.experimental import pallas as pl
from jax.experimental.pallas import tpu as pltpu
from jax.experimental.pallas import tpu_sc as plsc

N = 10000
NPAD = 10240
D = 128
N_EDGES_TOTAL = 330000
NW = 32
EW = 10368
EPAD = NW * EW
G = 128
ROWB = 1024
NBLK = NPAD // ROWB


def _act(p0, p1, b):
    g = jnp.maximum(p0 + p1 + b, 0.0)
    return jnp.maximum(g, 0.3 * g)


def _tc_first_body(x_ref, w_ref, attp_ref, h_ref, at_ref):
    g = x_ref[...]
    h = lax.dot_general(g, w_ref[...], (((1,), (1,)), ((), ())),
                        preferred_element_type=jnp.float32)
    h_ref[...] = h
    at_ref[...] = lax.dot_general(attp_ref[...], h, (((1,), (1,)), ((), ())),
                                  preferred_element_type=jnp.float32)


def _tc_mid_body(p0_ref, p1_ref, b_ref, w_ref, attp_ref, h_ref, at_ref):
    g = _act(p0_ref[...], p1_ref[...], b_ref[...])
    h = lax.dot_general(g, w_ref[...], (((1,), (1,)), ((), ())),
                        preferred_element_type=jnp.float32)
    h_ref[...] = h
    at_ref[...] = lax.dot_general(attp_ref[...], h, (((1,), (1,)), ((), ())),
                                  preferred_element_type=jnp.float32)


def _tc_last_body(p0_ref, p1_ref, b_ref, ow_ref, ob_ref, xe_ref, lg_ref, yp_ref):
    xe = _act(p0_ref[...], p1_ref[...], b_ref[...])
    xe_ref[...] = xe
    lg = lax.dot_general(xe, ow_ref[...], (((1,), (1,)), ((), ())),
                         preferred_element_type=jnp.float32) + ob_ref[...]
    lg_ref[...] = lg
    l0 = lg[:, 0]
    l1 = lg[:, 1]
    l2 = lg[:, 2]
    yp = jnp.where((l0 >= l1) & (l0 >= l2), 0, jnp.where(l1 >= l2, 1, 2))
    yp_ref[...] = jnp.zeros(yp_ref.shape, jnp.int32)
    yp_ref[0, :] = yp.astype(jnp.int32)


_row_spec = pl.BlockSpec((ROWB, D), lambda i: (i, 0))
_full_spec = pl.BlockSpec((D, D), lambda i: (0, 0))
_attp_spec = pl.BlockSpec((8, D), lambda i: (0, 0))
_b_spec = pl.BlockSpec((1, D), lambda i: (0, 0))
_at_spec = pl.BlockSpec((8, ROWB), lambda i: (0, i))

_h_at_out = (jax.ShapeDtypeStruct((NPAD, D), jnp.float32),
             jax.ShapeDtypeStruct((8, NPAD), jnp.float32))

_tc_first = pl.pallas_call(
    _tc_first_body, grid=(NBLK,),
    in_specs=[_row_spec, _full_spec, _attp_spec],
    out_specs=(_row_spec, _at_spec),
    out_shape=_h_at_out)

_tc_mid = pl.pallas_call(
    _tc_mid_body, grid=(NBLK,),
    in_specs=[_row_spec, _row_spec, _b_spec, _full_spec, _attp_spec],
    out_specs=(_row_spec, _at_spec),
    out_shape=_h_at_out)

_tc_last = pl.pallas_call(
    _tc_last_body, grid=(NBLK,),
    in_specs=[_row_spec, _row_spec, _b_spec, _full_spec, _b_spec],
    out_specs=(_row_spec, _row_spec, pl.BlockSpec((8, ROWB), lambda i: (0, i))),
    out_shape=(jax.ShapeDtypeStruct((NPAD, D), jnp.float32),
               jax.ShapeDtypeStruct((NPAD, D), jnp.float32),
               jax.ShapeDtypeStruct((8, NPAD), jnp.int32)))


_mesh = plsc.VectorSubcoreMesh(core_axis_name="c", subcore_axis_name="s")


def _sc_alpha_body(at_hbm, src_hbm, dst_hbm, ex_hbm, den_hbm,
                   ai_v, aj_v, den_v, src_v, dst_v, ex_v):
    cid = lax.axis_index("c")
    sid = lax.axis_index("s")
    wid = sid * 2 + cid
    base = wid * EW
    pltpu.sync_copy(at_hbm.at[0], ai_v)
    pltpu.sync_copy(at_hbm.at[1], aj_v)
    pltpu.sync_copy(src_hbm.at[pl.ds(base, EW)], src_v)
    pltpu.sync_copy(dst_hbm.at[pl.ds(base, EW)], dst_v)

    def mx(k, acc):
        return jnp.maximum(acc, ai_v[pl.ds(k * 16, 16)])
    acc = lax.fori_loop(0, NPAD // 16, mx, ai_v[pl.ds(0, 16)])
    lanes = lax.iota(jnp.int32, 16)
    for sh in (8, 4, 2, 1):
        ex_v[pl.ds(0, 16)] = acc
        acc = jnp.maximum(acc, plsc.load_gather(ex_v, [lanes ^ sh]))
    amax = acc

    zero16 = jnp.zeros((16,), jnp.float32)

    def zz(k, _):
        den_v[pl.ds(k * 16, 16)] = zero16
        return 0
    lax.fori_loop(0, NPAD // 16, zz, 0)

    def step(k, _):
        s16 = src_v[pl.ds(k * 16, 16)]
        d16 = dst_v[pl.ds(k * 16, 16)]
        ai = plsc.load_gather(ai_v, [d16])
        aj = plsc.load_gather(aj_v, [s16])
        ssum = ai + aj
        z = jnp.maximum(ssum, 0.2 * ssum)
        t = amax + aj
        c = jnp.maximum(t, 0.2 * t)
        ex = jnp.exp(z - c)
        ex_v[pl.ds(k * 16, 16)] = ex
        plsc.addupdate_scatter(den_v, [s16], ex)
        return 0
    lax.fori_loop(0, EW // 16, step, 0)

    pltpu.sync_copy(ex_v, ex_hbm.at[pl.ds(base, EW)])
    pltpu.sync_copy(den_v, den_hbm.at[wid])


_sc_alpha = pl.kernel(
    _sc_alpha_body,
    out_type=(jax.ShapeDtypeStruct((EPAD,), jnp.float32),
              jax.ShapeDtypeStruct((NW, NPAD), jnp.float32)),
    mesh=_mesh,
    compiler_params=pltpu.CompilerParams(needs_layout_passes=False),
    scratch_types=[
        pltpu.VMEM((NPAD,), jnp.float32),
        pltpu.VMEM((NPAD,), jnp.float32),
        pltpu.VMEM((NPAD,), jnp.float32),
        pltpu.VMEM((EW,), jnp.int32),
        pltpu.VMEM((EW,), jnp.int32),
        pltpu.VMEM((EW,), jnp.float32),
    ])


def _sc_agg_body(h_hbm, ex_hbm, src_hbm, dst_hbm, den_hbm, out_hbm,
                 den_v, stg_v, sbuf, dbuf, exbuf, wbuf, rows_v, out_sh, sem):
    cid = lax.axis_index("c")
    sid = lax.axis_index("s")
    wid = sid * 2 + cid
    base = wid * EW

    pltpu.sync_copy(den_hbm.at[0], den_v)

    def accum(r, _):
        pltpu.sync_copy(den_hbm.at[r], stg_v)

        def add16(k, _):
            i = pl.ds(k * 16, 16)
            den_v[i] = den_v[i] + stg_v[i]
            return 0
        lax.fori_loop(0, NPAD // 16, add16, 0)
        return 0
    lax.fori_loop(1, NW, accum, 0)

    def rcp(k, _):
        i = pl.ds(k * 16, 16)
        den_v[i] = 1.0 / (den_v[i] + 1e-16)
        return 0
    lax.fori_loop(0, NPAD // 16, rcp, 0)

    zero16 = jnp.zeros((16,), jnp.float32)

    def zrow(k, _):
        r = k // 8
        c = k % 8
        rows_v[r, pl.ds(c * 16, 16)] = zero16
        return 0
    lax.fori_loop(0, G * 8, zrow, 0)
    for j in range(NPAD // 16 // G):
        pltpu.sync_copy(rows_v, out_sh.at[pl.ds(sid * (NPAD // 16) + j * G, G)])
    plsc.subcore_barrier()

    def chunk(j, _):
        eb = base + j * G
        pltpu.sync_copy(src_hbm.at[pl.ds(eb, G)], sbuf)
        pltpu.sync_copy(dst_hbm.at[pl.ds(eb, G)], dbuf)
        pltpu.sync_copy(ex_hbm.at[pl.ds(eb, G)], exbuf)
        pltpu.async_copy(h_hbm.at[sbuf], rows_v, sem).wait()

        def wk(k, _):
            i = pl.ds(k * 16, 16)
            s16 = sbuf[i]
            wbuf[i] = exbuf[i] * plsc.load_gather(den_v, [s16])
            return 0
        lax.fori_loop(0, G // 16, wk, 0)

        def scale(r, _):
            wb = plsc.load_gather(wbuf, [jnp.full((16,), r, jnp.int32)])
            for c in range(8):
                i = pl.ds(c * 16, 16)
                rows_v[r, i] = rows_v[r, i] * wb
            return 0
        lax.fori_loop(0, G, scale, 0)
        pltpu.sync_copy(rows_v, out_sh.at[dbuf], add=True)
        return 0
    lax.fori_loop(0, EW // G, chunk, 0)
    plsc.subcore_barrier()

    for j in range(NPAD // 16 // G):
        r0 = sid * (NPAD // 16) + j * G
        pltpu.sync_copy(out_sh.at[pl.ds(r0, G)], out_hbm.at[cid, pl.ds(r0, G)])


_sc_agg = pl.kernel(
    _sc_agg_body,
    out_type=jax.ShapeDtypeStruct((2, NPAD, D), jnp.float32),
    mesh=_mesh,
    compiler_params=pltpu.CompilerParams(needs_layout_passes=False),
    scratch_types=[
        pltpu.VMEM((NPAD,), jnp.float32),
        pltpu.VMEM((NPAD,), jnp.float32),
        pltpu.VMEM((G,), jnp.int32),
        pltpu.VMEM((G,), jnp.int32),
        pltpu.VMEM((G,), jnp.float32),
        pltpu.VMEM((G,), jnp.float32),
        pltpu.VMEM((G, D), jnp.float32),
        pltpu.VMEM_SHARED((NPAD, D), jnp.float32),
        pltpu.SemaphoreType.DMA,
    ])


def _attp(att):
    a = jnp.zeros((8, D), jnp.float32)
    return a.at[0].set(att[0, 0, :D]).at[1].set(att[0, 0, D:])


def kernel(x, edge_index, node_label, node_index,
           W1, att1, b1, W2, att2, b2, W3, att3, b3, outW, outb):
    del node_label
    x_pad = jnp.zeros((NPAD, D), jnp.float32).at[:N].set(x)
    loops = jnp.arange(N, dtype=jnp.int32)
    padi = jnp.full((EPAD - N_EDGES_TOTAL,), NPAD - 1, jnp.int32)
    src = jnp.concatenate([edge_index[0], loops, padi])
    dst = jnp.concatenate([edge_index[1], loops, padi])

    def layer(g_parts, W, att, first):
        attp = _attp(att)
        if first:
            h, aT = _tc_first(g_parts, W, attp)
        else:
            p, b_prev = g_parts
            h, aT = _tc_mid(p[0], p[1], b_prev.reshape(1, D), W, attp)
        ex, den = _sc_alpha(aT, src, dst)
        return _sc_agg(h, ex, src, dst, den), h

    o1, _ = layer(x_pad, W1, att1, True)
    o2, _ = layer((o1, b1), W2, att2, False)
    o3, _ = layer((o2, b2), W3, att3, False)

    owp = jnp.zeros((D, D), jnp.float32).at[:3].set(outW)
    obp = jnp.zeros((1, D), jnp.float32).at[0, :3].set(outb)
    xe, lg, ypr = _tc_last(o3[0], o3[1], b3.reshape(1, D), owp, obp)

    x_embed = xe[:N]
    output = lg[:N, :3]
    ypred = ypr[0, :N]
    node_output = output[node_index]
    y_nodepred = ypred[node_index]
    return (x_embed, node_output, ypred, y_nodepred)


# speedup vs baseline: 12.7499x; 1.1621x over previous
import functools

import jax
import jax.numpy as jnp
from jax import lax
from jax.experimental import pallas as pl
from jax.experimental.pallas import tpu as pltpu
from jax.experimental.pallas import tpu_sc as plsc

N = 10000
NPAD = 10240
D = 128
N_EDGES_TOTAL = 330000
NW = 32
G = 32
NCH = 324
EW = NCH * G
EPAD = NW * EW
ROWB = 1024
NBLK = NPAD // ROWB


def _act(p0, p1, b):
    g = jnp.maximum(p0 + p1 + b, 0.0)
    return jnp.maximum(g, 0.3 * g)


def _tc_first_body(x_ref, w_ref, attp_ref, h_ref, at_ref):
    g = x_ref[...]
    h = lax.dot_general(g, w_ref[...], (((1,), (1,)), ((), ())),
                        preferred_element_type=jnp.float32)
    h_ref[...] = h
    at_ref[...] = lax.dot_general(attp_ref[...], h, (((1,), (1,)), ((), ())),
                                  preferred_element_type=jnp.float32)


def _tc_mid_body(p0_ref, p1_ref, b_ref, w_ref, attp_ref, h_ref, at_ref):
    g = _act(p0_ref[...], p1_ref[...], b_ref[...])
    h = lax.dot_general(g, w_ref[...], (((1,), (1,)), ((), ())),
                        preferred_element_type=jnp.float32)
    h_ref[...] = h
    at_ref[...] = lax.dot_general(attp_ref[...], h, (((1,), (1,)), ((), ())),
                                  preferred_element_type=jnp.float32)


def _tc_last_body(p0_ref, p1_ref, b_ref, ow_ref, ob_ref, xe_ref, lg_ref, yp_ref):
    xe = _act(p0_ref[...], p1_ref[...], b_ref[...])
    xe_ref[...] = xe
    lg = lax.dot_general(xe, ow_ref[...], (((1,), (1,)), ((), ())),
                         preferred_element_type=jnp.float32) + ob_ref[...]
    lg_ref[...] = lg
    l0 = lg[:, 0]
    l1 = lg[:, 1]
    l2 = lg[:, 2]
    yp = jnp.where((l0 >= l1) & (l0 >= l2), 0, jnp.where(l1 >= l2, 1, 2))
    yp_ref[...] = jnp.zeros(yp_ref.shape, jnp.int32)
    yp_ref[0, :] = yp.astype(jnp.int32)


_row_spec = pl.BlockSpec((ROWB, D), lambda i: (i, 0))
_full_spec = pl.BlockSpec((D, D), lambda i: (0, 0))
_attp_spec = pl.BlockSpec((8, D), lambda i: (0, 0))
_b_spec = pl.BlockSpec((1, D), lambda i: (0, 0))
_at_spec = pl.BlockSpec((8, ROWB), lambda i: (0, i))

_h_at_out = (jax.ShapeDtypeStruct((NPAD, D), jnp.float32),
             jax.ShapeDtypeStruct((8, NPAD), jnp.float32))

_tc_first = pl.pallas_call(
    _tc_first_body, grid=(NBLK,),
    in_specs=[_row_spec, _full_spec, _attp_spec],
    out_specs=(_row_spec, _at_spec),
    out_shape=_h_at_out)

_tc_mid = pl.pallas_call(
    _tc_mid_body, grid=(NBLK,),
    in_specs=[_row_spec, _row_spec, _b_spec, _full_spec, _attp_spec],
    out_specs=(_row_spec, _at_spec),
    out_shape=_h_at_out)

_tc_last = pl.pallas_call(
    _tc_last_body, grid=(NBLK,),
    in_specs=[_row_spec, _row_spec, _b_spec, _full_spec, _b_spec],
    out_specs=(_row_spec, _row_spec, pl.BlockSpec((8, ROWB), lambda i: (0, i))),
    out_shape=(jax.ShapeDtypeStruct((NPAD, D), jnp.float32),
               jax.ShapeDtypeStruct((NPAD, D), jnp.float32),
               jax.ShapeDtypeStruct((8, NPAD), jnp.int32)))


_mesh = plsc.VectorSubcoreMesh(core_axis_name="c", subcore_axis_name="s")


def _sc_alpha_body(at_hbm, src_hbm, dst_hbm, ex_hbm, den_hbm,
                   ai_v, aj_v, den_v, idi_v, src_v, dst_v, ex_v, den_sh):
    cid = lax.axis_index("c")
    sid = lax.axis_index("s")
    wid = sid * 2 + cid
    base = wid * EW
    pltpu.sync_copy(at_hbm.at[0], ai_v)
    pltpu.sync_copy(at_hbm.at[1], aj_v)
    pltpu.sync_copy(src_hbm.at[pl.ds(base, EW)], src_v)
    pltpu.sync_copy(dst_hbm.at[pl.ds(base, EW)], dst_v)

    def mx(k, acc):
        return jnp.maximum(acc, ai_v[pl.ds(k * 16, 16)])
    acc = lax.fori_loop(0, NPAD // 16, mx, ai_v[pl.ds(0, 16)])
    lanes = lax.iota(jnp.int32, 16)
    for sh in (8, 4, 2, 1):
        ex_v[pl.ds(0, 16)] = acc
        acc = jnp.maximum(acc, plsc.load_gather(ex_v, [lanes ^ sh]))
    amax = acc

    for m in range(5):
        idi_v[0, pl.ds(m * 16, 16)] = lanes + m * 16

    zero16 = jnp.zeros((16,), jnp.float32)

    def zz(k, _):
        den_v[k // 8, pl.ds((k % 8) * 16, 16)] = zero16
        return 0
    lax.fori_loop(0, NPAD // 16, zz, 0)

    @pl.when(sid == 0)
    def _():
        pltpu.sync_copy(den_v, den_sh)
    plsc.subcore_barrier()

    def step(k, _):
        s16 = src_v[pl.ds(k * 16, 16)]
        d16 = dst_v[pl.ds(k * 16, 16)]
        ai = plsc.load_gather(ai_v, [d16])
        aj = plsc.load_gather(aj_v, [s16])
        ssum = ai + aj
        z = jnp.maximum(ssum, 0.2 * ssum)
        t = amax + aj
        c = jnp.maximum(t, 0.2 * t)
        ex = jnp.exp(z - c)
        ex_v[pl.ds(k * 16, 16)] = ex
        plsc.addupdate_scatter(
            den_v, [lax.shift_right_logical(s16, 7), jnp.bitwise_and(s16, 127)], ex)
        return 0
    lax.fori_loop(0, EW // 16, step, 0)

    pltpu.sync_copy(ex_v, ex_hbm.at[pl.ds(base, EW)])
    pltpu.sync_copy(den_v, den_sh.at[idi_v.at[0]], add=True)
    plsc.subcore_barrier()

    @pl.when(sid < 10)
    def _():
        pltpu.sync_copy(den_sh.at[pl.ds(sid * 8, 8)],
                        den_hbm.at[cid, pl.ds(sid * 8, 8)])


_sc_alpha = pl.kernel(
    _sc_alpha_body,
    out_type=(jax.ShapeDtypeStruct((EPAD,), jnp.float32),
              jax.ShapeDtypeStruct((2, NPAD // 128, 128), jnp.float32)),
    mesh=_mesh,
    compiler_params=pltpu.CompilerParams(needs_layout_passes=False),
    scratch_types=[
        pltpu.VMEM((NPAD,), jnp.float32),
        pltpu.VMEM((NPAD,), jnp.float32),
        pltpu.VMEM((NPAD // 128, 128), jnp.float32),
        pltpu.VMEM((1, NPAD // 128), jnp.int32),
        pltpu.VMEM((EW,), jnp.int32),
        pltpu.VMEM((EW,), jnp.int32),
        pltpu.VMEM((EW,), jnp.float32),
        pltpu.VMEM_SHARED((NPAD // 128, 128), jnp.float32),
    ])


def _sc_agg_body(h_hbm, de_hbm, src_hbm, den_hbm, out_hbm,
                 den_v, src2, de0, de1, dx0, dx1, wbuf,
                 rows0, rows1, srows0, srows1, out_sh,
                 g0, g1, s0, s1, e0, e1):
    cid = lax.axis_index("c")
    sid = lax.axis_index("s")
    wid = sid * 2 + cid
    rows = (rows0, rows1)
    srows = (srows0, srows1)
    de = (de0, de1)
    dx = (dx0, dx1)
    gsem = (g0, g1)
    ssem = (s0, s1)
    esem = (e0, e1)

    pltpu.sync_copy(src_hbm.at[wid], src2)

    def src_at(c):
        return src2.at[lax.shift_right_logical(c, 2),
                       pl.ds(jnp.bitwise_and(c, 3) * G, G)]

    pltpu.sync_copy(den_hbm.at[0], den_v)
    for buf, off, nn in ((rows0, 0, 32), (rows1, 32, 32), (srows0, 64, 16)):
        pltpu.sync_copy(den_hbm.at[1, pl.ds(off, nn)], buf.at[pl.ds(0, nn)])

        def rcp(k, _, buf=buf, off=off):
            r, i = k // 8, pl.ds((k % 8) * 16, 16)
            den_v[off + r, i] = 1.0 / (den_v[off + r, i] + buf[r, i] + 1e-16)
            return 0
        lax.fori_loop(0, nn * 8, rcp, 0)

    zero16 = jnp.zeros((16,), jnp.float32)

    def zrow(k, _):
        srows0[k // 8, pl.ds((k % 8) * 16, 16)] = zero16
        return 0
    lax.fori_loop(0, G * 8, zrow, 0)
    nz = (NPAD // 16 + G - 1) // G
    for j in range(nz - 1):
        pltpu.sync_copy(srows0, out_sh.at[pl.ds(sid * (NPAD // 16) + j * G, G)])
    rem = NPAD // 16 - (nz - 1) * G
    pltpu.sync_copy(srows0.at[pl.ds(0, rem)],
                    out_sh.at[pl.ds(sid * (NPAD // 16) + (nz - 1) * G, rem)])
    plsc.subcore_barrier()

    pltpu.async_copy(de_hbm.at[wid, 0], de0, e0)
    pltpu.async_copy(de_hbm.at[wid, 1], de1, e1)
    pltpu.async_copy(h_hbm.at[src_at(0)], rows0, g0)
    pltpu.async_copy(h_hbm.at[src_at(1)], rows1, g1)

    def do_slot(j, c, p):
        pltpu.make_async_copy(de_hbm.at[wid, c], de[p], esem[p]).wait()
        pltpu.make_async_copy(h_hbm.at[src_at(c)], rows[p], gsem[p]).wait()

        cr = lax.shift_right_logical(c, 2)
        cc0 = jnp.bitwise_and(c, 3) * G
        for k in range(G // 16):
            i = pl.ds(cc0 + k * 16, 16)
            s16 = src2[cr, i]
            ex16 = plsc.bitcast(de[p][1, pl.ds(k * 16, 16)], jnp.float32)
            wbuf[pl.ds(k * 16, 16)] = ex16 * plsc.load_gather(
                den_v, [lax.shift_right_logical(s16, 7),
                        jnp.bitwise_and(s16, 127)])

        @pl.when(j > 0)
        def _():
            pltpu.make_async_copy(srows[p], out_sh.at[dx[p].at[0]],
                                  ssem[p]).wait()
        for k in range(G // 16):
            i = pl.ds(k * 16, 16)
            dx[p][0, i] = de[p][0, i]

        def scale(r, _):
            wb = plsc.load_gather(wbuf, [jnp.full((16,), r, jnp.int32)])
            for cc in range(8):
                i = pl.ds(cc * 16, 16)
                srows[p][r, i] = rows[p][r, i] * wb
            return 0
        lax.fori_loop(0, G, scale, 0)
        pltpu.async_copy(srows[p], out_sh.at[dx[p].at[0]], ssem[p], add=True)

        @pl.when(j < NCH // 2 - 1)
        def _():
            pltpu.async_copy(de_hbm.at[wid, c + 2], de[p], esem[p])
            pltpu.async_copy(h_hbm.at[src_at(c + 2)], rows[p], gsem[p])

    def pair(j, _):
        do_slot(j, 2 * j, 0)
        do_slot(j, 2 * j + 1, 1)
        return 0
    lax.fori_loop(0, NCH // 2, pair, 0)
    pltpu.make_async_copy(srows0, out_sh.at[dx0.at[0]], s0).wait()
    pltpu.make_async_copy(srows1, out_sh.at[dx1.at[0]], s1).wait()
    plsc.subcore_barrier()

    for j in range(nz - 1):
        r0 = sid * (NPAD // 16) + j * G
        pltpu.sync_copy(out_sh.at[pl.ds(r0, G)], out_hbm.at[cid, pl.ds(r0, G)])
    r0 = sid * (NPAD // 16) + (nz - 1) * G
    pltpu.sync_copy(out_sh.at[pl.ds(r0, rem)], out_hbm.at[cid, pl.ds(r0, rem)])


_sc_agg = pl.kernel(
    _sc_agg_body,
    out_type=jax.ShapeDtypeStruct((2, NPAD, D), jnp.float32),
    mesh=_mesh,
    compiler_params=pltpu.CompilerParams(needs_layout_passes=False),
    scratch_types=[
        pltpu.VMEM((NPAD // 128, 128), jnp.float32),
        pltpu.VMEM((NCH // 4, 4 * G), jnp.int32),
        pltpu.VMEM((2, G), jnp.int32),
        pltpu.VMEM((2, G), jnp.int32),
        pltpu.VMEM((1, G), jnp.int32),
        pltpu.VMEM((1, G), jnp.int32),
        pltpu.VMEM((G,), jnp.float32),
        pltpu.VMEM((G, D), jnp.float32),
        pltpu.VMEM((G, D), jnp.float32),
        pltpu.VMEM((G, D), jnp.float32),
        pltpu.VMEM((G, D), jnp.float32),
        pltpu.VMEM_SHARED((NPAD, D), jnp.float32),
        pltpu.SemaphoreType.DMA,
        pltpu.SemaphoreType.DMA,
        pltpu.SemaphoreType.DMA,
        pltpu.SemaphoreType.DMA,
        pltpu.SemaphoreType.DMA,
        pltpu.SemaphoreType.DMA,
    ])


def _attp(att):
    a = jnp.zeros((8, D), jnp.float32)
    return a.at[0].set(att[0, 0, :D]).at[1].set(att[0, 0, D:])


def kernel(x, edge_index, node_label, node_index,
           W1, att1, b1, W2, att2, b2, W3, att3, b3, outW, outb):
    del node_label
    x_pad = jnp.zeros((NPAD, D), jnp.float32).at[:N].set(x)
    loops = jnp.arange(N, dtype=jnp.int32)
    padi = jnp.full((EPAD - N_EDGES_TOTAL,), NPAD - 1, jnp.int32)
    src = jnp.concatenate([edge_index[0], loops, padi])
    dst = jnp.concatenate([edge_index[1], loops, padi])
    src3 = src.reshape(NW, NCH // 4, 4 * G)
    dst3 = dst.reshape(NW, NCH, G)

    def layer(g_parts, W, att, first):
        attp = _attp(att)
        if first:
            h, aT = _tc_first(g_parts, W, attp)
        else:
            p, b_prev = g_parts
            h, aT = _tc_mid(p[0], p[1], b_prev.reshape(1, D), W, attp)
        ex, den = _sc_alpha(aT, src, dst)
        de3 = jnp.stack([dst3, lax.bitcast_convert_type(
            ex.reshape(NW, NCH, G), jnp.int32)], axis=2)
        return _sc_agg(h, de3, src3, den), h

    o1, _ = layer(x_pad, W1, att1, True)
    o2, _ = layer((o1, b1), W2, att2, False)
    o3, _ = layer((o2, b2), W3, att3, False)

    owp = jnp.zeros((D, D), jnp.float32).at[:3].set(outW)
    obp = jnp.zeros((1, D), jnp.float32).at[0, :3].set(outb)
    xe, lg, ypr = _tc_last(o3[0], o3[1], b3.reshape(1, D), owp, obp)

    x_embed = xe[:N]
    output = lg[:N, :3]
    ypred = ypr[0, :N]
    node_output = output[node_index]
    y_nodepred = ypred[node_index]
    return (x_embed, node_output, ypred, y_nodepred)


# speedup vs baseline: 19.6496x; 1.5412x over previous
import functools

import jax
import jax.numpy as jnp
from jax import lax
from jax.experimental import pallas as pl
from jax.experimental.pallas import tpu as pltpu
from jax.experimental.pallas import tpu_sc as plsc

N = 10000
NPAD = 10240
D = 128
N_EDGES_TOTAL = 330000
NW = 32
G = 32
NCH = 324
EW = NCH * G
EPAD = NW * EW
ROWB = 1024
NBLK = NPAD // ROWB


def _act(p0, p1, b):
    g = jnp.maximum(p0 + p1 + b, 0.0)
    return jnp.maximum(g, 0.3 * g)


def _tc_first_body(x_ref, w_ref, attp_ref, h_ref, at_ref):
    g = x_ref[...]
    h = lax.dot_general(g, w_ref[...], (((1,), (1,)), ((), ())),
                        preferred_element_type=jnp.float32)
    h_ref[...] = h
    at_ref[...] = lax.dot_general(attp_ref[...], h, (((1,), (1,)), ((), ())),
                                  preferred_element_type=jnp.float32)


def _tc_mid_body(p0_ref, p1_ref, b_ref, w_ref, attp_ref, h_ref, at_ref):
    g = _act(p0_ref[...], p1_ref[...], b_ref[...])
    h = lax.dot_general(g, w_ref[...], (((1,), (1,)), ((), ())),
                        preferred_element_type=jnp.float32)
    h_ref[...] = h
    at_ref[...] = lax.dot_general(attp_ref[...], h, (((1,), (1,)), ((), ())),
                                  preferred_element_type=jnp.float32)


def _tc_last_body(p0_ref, p1_ref, b_ref, ow_ref, ob_ref, xe_ref, lg_ref, yp_ref):
    xe = _act(p0_ref[...], p1_ref[...], b_ref[...])
    xe_ref[...] = xe
    lg = lax.dot_general(xe, ow_ref[...], (((1,), (1,)), ((), ())),
                         preferred_element_type=jnp.float32) + ob_ref[...]
    lg_ref[...] = lg
    l0 = lg[:, 0]
    l1 = lg[:, 1]
    l2 = lg[:, 2]
    yp = jnp.where((l0 >= l1) & (l0 >= l2), 0, jnp.where(l1 >= l2, 1, 2))
    yp_ref[...] = jnp.zeros(yp_ref.shape, jnp.int32)
    yp_ref[0, :] = yp.astype(jnp.int32)


_row_spec = pl.BlockSpec((ROWB, D), lambda i: (i, 0))
_full_spec = pl.BlockSpec((D, D), lambda i: (0, 0))
_attp_spec = pl.BlockSpec((8, D), lambda i: (0, 0))
_b_spec = pl.BlockSpec((1, D), lambda i: (0, 0))
_at_spec = pl.BlockSpec((8, ROWB), lambda i: (0, i))

_h_at_out = (jax.ShapeDtypeStruct((NPAD, D), jnp.float32),
             jax.ShapeDtypeStruct((8, NPAD), jnp.float32))

_tc_first = pl.pallas_call(
    _tc_first_body, grid=(NBLK,),
    in_specs=[_row_spec, _full_spec, _attp_spec],
    out_specs=(_row_spec, _at_spec),
    out_shape=_h_at_out)

_tc_mid = pl.pallas_call(
    _tc_mid_body, grid=(NBLK,),
    in_specs=[_row_spec, _row_spec, _b_spec, _full_spec, _attp_spec],
    out_specs=(_row_spec, _at_spec),
    out_shape=_h_at_out)

_tc_last = pl.pallas_call(
    _tc_last_body, grid=(NBLK,),
    in_specs=[_row_spec, _row_spec, _b_spec, _full_spec, _b_spec],
    out_specs=(_row_spec, _row_spec, pl.BlockSpec((8, ROWB), lambda i: (0, i))),
    out_shape=(jax.ShapeDtypeStruct((NPAD, D), jnp.float32),
               jax.ShapeDtypeStruct((NPAD, D), jnp.float32),
               jax.ShapeDtypeStruct((8, NPAD), jnp.int32)))


_mesh = plsc.VectorSubcoreMesh(core_axis_name="c", subcore_axis_name="s")


def _sc_alpha_body(at_hbm, src_hbm, dst_hbm, ex_hbm, den_hbm,
                   ai_v, aj_v, den_v, idi_v, src_v, dst_v, ex_v, den_sh):
    cid = lax.axis_index("c")
    sid = lax.axis_index("s")
    wid = sid * 2 + cid
    base = wid * EW
    pltpu.sync_copy(at_hbm.at[0], ai_v)
    pltpu.sync_copy(at_hbm.at[1], aj_v)
    pltpu.sync_copy(src_hbm.at[pl.ds(base, EW)], src_v)
    pltpu.sync_copy(dst_hbm.at[pl.ds(base, EW)], dst_v)

    def mx(k, acc):
        return jnp.maximum(acc, ai_v[pl.ds(k * 16, 16)])
    acc = lax.fori_loop(0, NPAD // 16, mx, ai_v[pl.ds(0, 16)])
    lanes = lax.iota(jnp.int32, 16)
    for sh in (8, 4, 2, 1):
        ex_v[pl.ds(0, 16)] = acc
        acc = jnp.maximum(acc, plsc.load_gather(ex_v, [lanes ^ sh]))
    amax = acc

    for m in range(5):
        idi_v[0, pl.ds(m * 16, 16)] = lanes + m * 16

    zero16 = jnp.zeros((16,), jnp.float32)

    def zz(k, _):
        den_v[k // 8, pl.ds((k % 8) * 16, 16)] = zero16
        return 0
    lax.fori_loop(0, NPAD // 16, zz, 0)

    @pl.when(sid == 0)
    def _():
        pltpu.sync_copy(den_v, den_sh)
    plsc.subcore_barrier()

    def step(k, _):
        s16 = src_v[pl.ds(k * 16, 16)]
        d16 = dst_v[pl.ds(k * 16, 16)]
        ai = plsc.load_gather(ai_v, [d16])
        aj = plsc.load_gather(aj_v, [s16])
        ssum = ai + aj
        z = jnp.maximum(ssum, 0.2 * ssum)
        t = amax + aj
        c = jnp.maximum(t, 0.2 * t)
        ex = jnp.exp(z - c)
        ex_v[pl.ds(k * 16, 16)] = ex
        plsc.addupdate_scatter(
            den_v, [lax.shift_right_logical(s16, 7), jnp.bitwise_and(s16, 127)], ex)
        return 0
    lax.fori_loop(0, EW // 16, step, 0)

    pltpu.sync_copy(ex_v, ex_hbm.at[pl.ds(base, EW)])
    pltpu.sync_copy(den_v, den_sh.at[idi_v.at[0]], add=True)
    plsc.subcore_barrier()

    @pl.when(sid < 10)
    def _():
        pltpu.sync_copy(den_sh.at[pl.ds(sid * 8, 8)],
                        den_hbm.at[cid, pl.ds(sid * 8, 8)])


_sc_alpha = pl.kernel(
    _sc_alpha_body,
    out_type=(jax.ShapeDtypeStruct((EPAD,), jnp.float32),
              jax.ShapeDtypeStruct((2, NPAD // 128, 128), jnp.float32)),
    mesh=_mesh,
    compiler_params=pltpu.CompilerParams(needs_layout_passes=False),
    scratch_types=[
        pltpu.VMEM((NPAD,), jnp.float32),
        pltpu.VMEM((NPAD,), jnp.float32),
        pltpu.VMEM((NPAD // 128, 128), jnp.float32),
        pltpu.VMEM((1, NPAD // 128), jnp.int32),
        pltpu.VMEM((EW,), jnp.int32),
        pltpu.VMEM((EW,), jnp.int32),
        pltpu.VMEM((EW,), jnp.float32),
        pltpu.VMEM_SHARED((NPAD // 128, 128), jnp.float32),
    ])


def _sc_agg_body(h_hbm, de_hbm, src_hbm, den_hbm, out_hbm,
                 den_v, src2, de0, de1, dx0, dx1, dx2, dx3, wbuf,
                 rows0, rows1, srows0, srows1, srows2, srows3, out_sh,
                 g0, g1, s0, s1, s2, s3, e0, e1):
    cid = lax.axis_index("c")
    sid = lax.axis_index("s")
    wid = sid * 2 + cid
    rows = (rows0, rows1)
    srows = (srows0, srows1, srows2, srows3)
    de = (de0, de1)
    dx = (dx0, dx1, dx2, dx3)
    gsem = (g0, g1)
    ssem = (s0, s1, s2, s3)
    esem = (e0, e1)

    pltpu.sync_copy(src_hbm.at[wid], src2)

    def src_at(c):
        return src2.at[lax.shift_right_logical(c, 2),
                       pl.ds(jnp.bitwise_and(c, 3) * G, G)]

    pltpu.sync_copy(den_hbm.at[0], den_v)
    for buf, off, nn in ((rows0, 0, 32), (rows1, 32, 32), (srows0, 64, 16)):
        pltpu.sync_copy(den_hbm.at[1, pl.ds(off, nn)], buf.at[pl.ds(0, nn)])

        def rcp(k, _, buf=buf, off=off):
            r, i = k // 8, pl.ds((k % 8) * 16, 16)
            den_v[off + r, i] = 1.0 / (den_v[off + r, i] + buf[r, i] + 1e-16)
            return 0
        lax.fori_loop(0, nn * 8, rcp, 0)

    zero16 = jnp.zeros((16,), jnp.float32)

    def zrow(k, _):
        srows0[k // 8, pl.ds((k % 8) * 16, 16)] = zero16
        return 0
    lax.fori_loop(0, G * 8, zrow, 0)
    nz = (NPAD // 16 + G - 1) // G
    for j in range(nz - 1):
        pltpu.sync_copy(srows0, out_sh.at[pl.ds(sid * (NPAD // 16) + j * G, G)])
    rem = NPAD // 16 - (nz - 1) * G
    pltpu.sync_copy(srows0.at[pl.ds(0, rem)],
                    out_sh.at[pl.ds(sid * (NPAD // 16) + (nz - 1) * G, rem)])
    plsc.subcore_barrier()

    pltpu.async_copy(de_hbm.at[wid, 0], de0, e0)
    pltpu.async_copy(de_hbm.at[wid, 1], de1, e1)
    pltpu.async_copy(h_hbm.at[src_at(0)], rows0, g0)
    pltpu.async_copy(h_hbm.at[src_at(1)], rows1, g1)

    def do_slot(j, c, p, q):
        pltpu.make_async_copy(de_hbm.at[wid, c], de[p], esem[p]).wait()
        pltpu.make_async_copy(h_hbm.at[src_at(c)], rows[p], gsem[p]).wait()

        cr = lax.shift_right_logical(c, 2)
        cc0 = jnp.bitwise_and(c, 3) * G
        for k in range(G // 16):
            i = pl.ds(cc0 + k * 16, 16)
            s16 = src2[cr, i]
            ex16 = plsc.bitcast(de[p][1, pl.ds(k * 16, 16)], jnp.float32)
            wbuf[pl.ds(k * 16, 16)] = ex16 * plsc.load_gather(
                den_v, [lax.shift_right_logical(s16, 7),
                        jnp.bitwise_and(s16, 127)])

        @pl.when(j > 0)
        def _():
            pltpu.make_async_copy(srows[q], out_sh.at[dx[q].at[0]],
                                  ssem[q]).wait()
        for k in range(G // 16):
            i = pl.ds(k * 16, 16)
            dx[q][0, i] = de[p][0, i]

        @plsc.parallel_loop(0, G, unroll=4)
        def scale(r):
            wb = plsc.load_gather(wbuf, [jnp.full((16,), r, jnp.int32)])
            for cc in range(8):
                i = pl.ds(cc * 16, 16)
                srows[q][r, i] = rows[p][r, i] * wb
        pltpu.async_copy(srows[q], out_sh.at[dx[q].at[0]], ssem[q], add=True)

        @pl.when(c + 2 < NCH)
        def _():
            pltpu.async_copy(de_hbm.at[wid, c + 2], de[p], esem[p])
            pltpu.async_copy(h_hbm.at[src_at(c + 2)], rows[p], gsem[p])

    def quad(j, _):
        for k in range(4):
            do_slot(j, 4 * j + k, k % 2, k)
        return 0
    lax.fori_loop(0, NCH // 4, quad, 0)
    for q in range(4):
        pltpu.make_async_copy(srows[q], out_sh.at[dx[q].at[0]], ssem[q]).wait()
    plsc.subcore_barrier()

    for j in range(nz - 1):
        r0 = sid * (NPAD // 16) + j * G
        pltpu.sync_copy(out_sh.at[pl.ds(r0, G)], out_hbm.at[cid, pl.ds(r0, G)])
    r0 = sid * (NPAD // 16) + (nz - 1) * G
    pltpu.sync_copy(out_sh.at[pl.ds(r0, rem)], out_hbm.at[cid, pl.ds(r0, rem)])


_sc_agg = pl.kernel(
    _sc_agg_body,
    out_type=jax.ShapeDtypeStruct((2, NPAD, D), jnp.float32),
    mesh=_mesh,
    compiler_params=pltpu.CompilerParams(needs_layout_passes=False),
    scratch_types=[
        pltpu.VMEM((NPAD // 128, 128), jnp.float32),
        pltpu.VMEM((NCH // 4, 4 * G), jnp.int32),
        pltpu.VMEM((2, G), jnp.int32),
        pltpu.VMEM((2, G), jnp.int32),
        pltpu.VMEM((1, G), jnp.int32),
        pltpu.VMEM((1, G), jnp.int32),
        pltpu.VMEM((1, G), jnp.int32),
        pltpu.VMEM((1, G), jnp.int32),
        pltpu.VMEM((G,), jnp.float32),
        pltpu.VMEM((G, D), jnp.float32),
        pltpu.VMEM((G, D), jnp.float32),
        pltpu.VMEM((G, D), jnp.float32),
        pltpu.VMEM((G, D), jnp.float32),
        pltpu.VMEM((G, D), jnp.float32),
        pltpu.VMEM((G, D), jnp.float32),
        pltpu.VMEM_SHARED((NPAD, D), jnp.float32),
        pltpu.SemaphoreType.DMA,
        pltpu.SemaphoreType.DMA,
        pltpu.SemaphoreType.DMA,
        pltpu.SemaphoreType.DMA,
        pltpu.SemaphoreType.DMA,
        pltpu.SemaphoreType.DMA,
        pltpu.SemaphoreType.DMA,
        pltpu.SemaphoreType.DMA,
    ])


def _attp(att):
    a = jnp.zeros((8, D), jnp.float32)
    return a.at[0].set(att[0, 0, :D]).at[1].set(att[0, 0, D:])


def kernel(x, edge_index, node_label, node_index,
           W1, att1, b1, W2, att2, b2, W3, att3, b3, outW, outb):
    del node_label
    x_pad = jnp.zeros((NPAD, D), jnp.float32).at[:N].set(x)
    loops = jnp.arange(N, dtype=jnp.int32)
    padi = jnp.full((EPAD - N_EDGES_TOTAL,), NPAD - 1, jnp.int32)
    src = jnp.concatenate([edge_index[0], loops, padi])
    dst = jnp.concatenate([edge_index[1], loops, padi])
    src3 = src.reshape(NW, NCH // 4, 4 * G)
    dst3 = dst.reshape(NW, NCH, G)

    def layer(g_parts, W, att, first):
        attp = _attp(att)
        if first:
            h, aT = _tc_first(g_parts, W, attp)
        else:
            p, b_prev = g_parts
            h, aT = _tc_mid(p[0], p[1], b_prev.reshape(1, D), W, attp)
        ex, den = _sc_alpha(aT, src, dst)
        de3 = jnp.stack([dst3, lax.bitcast_convert_type(
            ex.reshape(NW, NCH, G), jnp.int32)], axis=2)
        return _sc_agg(h, de3, src3, den), h

    o1, _ = layer(x_pad, W1, att1, True)
    o2, _ = layer((o1, b1), W2, att2, False)
    o3, _ = layer((o2, b2), W3, att3, False)

    owp = jnp.zeros((D, D), jnp.float32).at[:3].set(outW)
    obp = jnp.zeros((1, D), jnp.float32).at[0, :3].set(outb)
    xe, lg, ypr = _tc_last(o3[0], o3[1], b3.reshape(1, D), owp, obp)

    x_embed = xe[:N]
    output = lg[:N, :3]
    ypred = ypr[0, :N]
    node_output = output[node_index]
    y_nodepred = ypred[node_index]
    return (x_embed, node_output, ypred, y_nodepred)


# speedup vs baseline: 21.6929x; 1.1040x over previous
import functools

import jax
import jax.numpy as jnp
from jax import lax
from jax.experimental import pallas as pl
from jax.experimental.pallas import tpu as pltpu
from jax.experimental.pallas import tpu_sc as plsc

N = 10000
NPAD = 10240
D = 128
N_EDGES_TOTAL = 330000
NW = 32
G = 32
NCH = 324
EW = NCH * G
EPAD = NW * EW
ROWB = 1024
NBLK = NPAD // ROWB


def _act(p0, p1, b):
    g = jnp.maximum(p0 + p1 + b, 0.0)
    return jnp.maximum(g, 0.3 * g)


def _tc_first_body(x_ref, w_ref, attp_ref, h_ref, at_ref):
    g = x_ref[...]
    h = lax.dot_general(g, w_ref[...], (((1,), (1,)), ((), ())),
                        preferred_element_type=jnp.float32)
    h_ref[...] = h
    at_ref[...] = lax.dot_general(attp_ref[...], h, (((1,), (1,)), ((), ())),
                                  preferred_element_type=jnp.float32)


def _tc_mid_body(p0_ref, p1_ref, b_ref, w_ref, attp_ref, h_ref, at_ref):
    g = _act(p0_ref[...], p1_ref[...], b_ref[...])
    h = lax.dot_general(g, w_ref[...], (((1,), (1,)), ((), ())),
                        preferred_element_type=jnp.float32)
    h_ref[...] = h
    at_ref[...] = lax.dot_general(attp_ref[...], h, (((1,), (1,)), ((), ())),
                                  preferred_element_type=jnp.float32)


def _tc_last_body(p0_ref, p1_ref, b_ref, ow_ref, ob_ref, xe_ref, lg_ref, yp_ref):
    xe = _act(p0_ref[...], p1_ref[...], b_ref[...])
    xe_ref[...] = xe
    lg = lax.dot_general(xe, ow_ref[...], (((1,), (1,)), ((), ())),
                         preferred_element_type=jnp.float32) + ob_ref[...]
    lg_ref[...] = lg
    l0 = lg[:, 0]
    l1 = lg[:, 1]
    l2 = lg[:, 2]
    yp = jnp.where((l0 >= l1) & (l0 >= l2), 0, jnp.where(l1 >= l2, 1, 2))
    yp_ref[...] = jnp.zeros(yp_ref.shape, jnp.int32)
    yp_ref[0, :] = yp.astype(jnp.int32)


_row_spec = pl.BlockSpec((ROWB, D), lambda i: (i, 0))
_full_spec = pl.BlockSpec((D, D), lambda i: (0, 0))
_attp_spec = pl.BlockSpec((8, D), lambda i: (0, 0))
_b_spec = pl.BlockSpec((1, D), lambda i: (0, 0))
_at_spec = pl.BlockSpec((8, ROWB), lambda i: (0, i))

_h_at_out = (jax.ShapeDtypeStruct((NPAD, D), jnp.float32),
             jax.ShapeDtypeStruct((8, NPAD), jnp.float32))

_tc_first = pl.pallas_call(
    _tc_first_body, grid=(NBLK,),
    in_specs=[_row_spec, _full_spec, _attp_spec],
    out_specs=(_row_spec, _at_spec),
    out_shape=_h_at_out)

_tc_mid = pl.pallas_call(
    _tc_mid_body, grid=(NBLK,),
    in_specs=[_row_spec, _row_spec, _b_spec, _full_spec, _attp_spec],
    out_specs=(_row_spec, _at_spec),
    out_shape=_h_at_out)

_tc_last = pl.pallas_call(
    _tc_last_body, grid=(NBLK,),
    in_specs=[_row_spec, _row_spec, _b_spec, _full_spec, _b_spec],
    out_specs=(_row_spec, _row_spec, pl.BlockSpec((8, ROWB), lambda i: (0, i))),
    out_shape=(jax.ShapeDtypeStruct((NPAD, D), jnp.float32),
               jax.ShapeDtypeStruct((NPAD, D), jnp.float32),
               jax.ShapeDtypeStruct((8, NPAD), jnp.int32)))


_mesh = plsc.VectorSubcoreMesh(core_axis_name="c", subcore_axis_name="s")


def _sc_alpha_body(at_hbm, src_hbm, dst_hbm, ex_hbm, den_hbm,
                   ai_v, aj_v, den_v, idi_v, src_v, dst_v, ex_v, den_sh):
    cid = lax.axis_index("c")
    sid = lax.axis_index("s")
    wid = sid * 2 + cid
    base = wid * EW
    pltpu.sync_copy(at_hbm.at[0], ai_v)
    pltpu.sync_copy(at_hbm.at[1], aj_v)
    pltpu.sync_copy(src_hbm.at[pl.ds(base, EW)], src_v)
    pltpu.sync_copy(dst_hbm.at[pl.ds(base, EW)], dst_v)

    def mx(k, acc):
        return jnp.maximum(acc, ai_v[pl.ds(k * 16, 16)])
    acc = lax.fori_loop(0, NPAD // 16, mx, ai_v[pl.ds(0, 16)])
    lanes = lax.iota(jnp.int32, 16)
    for sh in (8, 4, 2, 1):
        ex_v[pl.ds(0, 16)] = acc
        acc = jnp.maximum(acc, plsc.load_gather(ex_v, [lanes ^ sh]))
    amax = acc

    for m in range(5):
        idi_v[0, pl.ds(m * 16, 16)] = lanes + m * 16

    zero16 = jnp.zeros((16,), jnp.float32)

    def zz(k, _):
        den_v[k // 8, pl.ds((k % 8) * 16, 16)] = zero16
        return 0
    lax.fori_loop(0, NPAD // 16, zz, 0)

    @pl.when(sid == 0)
    def _():
        pltpu.sync_copy(den_v, den_sh)
    plsc.subcore_barrier()

    def step(k, _):
        s16 = src_v[pl.ds(k * 16, 16)]
        d16 = dst_v[pl.ds(k * 16, 16)]
        ai = plsc.load_gather(ai_v, [d16])
        aj = plsc.load_gather(aj_v, [s16])
        ssum = ai + aj
        z = jnp.maximum(ssum, 0.2 * ssum)
        t = amax + aj
        c = jnp.maximum(t, 0.2 * t)
        ex = jnp.exp(z - c)
        ex_v[pl.ds(k * 16, 16)] = ex
        plsc.addupdate_scatter(
            den_v, [lax.shift_right_logical(s16, 7), jnp.bitwise_and(s16, 127)], ex)
        return 0
    lax.fori_loop(0, EW // 16, step, 0)

    pltpu.sync_copy(ex_v, ex_hbm.at[pl.ds(base, EW)])
    pltpu.sync_copy(den_v, den_sh.at[idi_v.at[0]], add=True)
    plsc.subcore_barrier()

    @pl.when(sid < 10)
    def _():
        pltpu.sync_copy(den_sh.at[pl.ds(sid * 8, 8)],
                        den_hbm.at[cid, pl.ds(sid * 8, 8)])


_sc_alpha = pl.kernel(
    _sc_alpha_body,
    out_type=(jax.ShapeDtypeStruct((EPAD,), jnp.float32),
              jax.ShapeDtypeStruct((2, NPAD // 128, 128), jnp.float32)),
    mesh=_mesh,
    compiler_params=pltpu.CompilerParams(needs_layout_passes=False),
    scratch_types=[
        pltpu.VMEM((NPAD,), jnp.float32),
        pltpu.VMEM((NPAD,), jnp.float32),
        pltpu.VMEM((NPAD // 128, 128), jnp.float32),
        pltpu.VMEM((1, NPAD // 128), jnp.int32),
        pltpu.VMEM((EW,), jnp.int32),
        pltpu.VMEM((EW,), jnp.int32),
        pltpu.VMEM((EW,), jnp.float32),
        pltpu.VMEM_SHARED((NPAD // 128, 128), jnp.float32),
    ])


def _sc_agg_body(h_hbm, ex_hbm, dst_hbm, src_hbm, den_hbm, out_hbm,
                 den_v, src2, db0, db1, xb0, xb1, dx0, dx1, dx2, dx3, wbuf,
                 rows0, rows1, srows0, srows1, srows2, srows3, out_sh,
                 g0, g1, s0, s1, s2, s3, d0, d1, x0, x1):
    cid = lax.axis_index("c")
    sid = lax.axis_index("s")
    wid = sid * 2 + cid
    rows = (rows0, rows1)
    srows = (srows0, srows1, srows2, srows3)
    db = (db0, db1)
    xb = (xb0, xb1)
    dx = (dx0, dx1, dx2, dx3)
    gsem = (g0, g1)
    ssem = (s0, s1, s2, s3)
    dsem = (d0, d1)
    xsem = (x0, x1)

    pltpu.sync_copy(src_hbm.at[wid], src2)

    def src_at(c):
        return src2.at[lax.shift_right_logical(c, 2),
                       pl.ds(jnp.bitwise_and(c, 3) * G, G)]

    pltpu.sync_copy(den_hbm.at[0], den_v)
    for buf, off, nn in ((rows0, 0, 32), (rows1, 32, 32), (srows0, 64, 16)):
        pltpu.sync_copy(den_hbm.at[1, pl.ds(off, nn)], buf.at[pl.ds(0, nn)])

        def rcp(k, _, buf=buf, off=off):
            r, i = k // 8, pl.ds((k % 8) * 16, 16)
            den_v[off + r, i] = 1.0 / (den_v[off + r, i] + buf[r, i] + 1e-16)
            return 0
        lax.fori_loop(0, nn * 8, rcp, 0)

    zero16 = jnp.zeros((16,), jnp.float32)

    def zrow(k, _):
        srows0[k // 8, pl.ds((k % 8) * 16, 16)] = zero16
        return 0
    lax.fori_loop(0, G * 8, zrow, 0)
    nz = (NPAD // 16 + G - 1) // G
    for j in range(nz - 1):
        pltpu.sync_copy(srows0, out_sh.at[pl.ds(sid * (NPAD // 16) + j * G, G)])
    rem = NPAD // 16 - (nz - 1) * G
    pltpu.sync_copy(srows0.at[pl.ds(0, rem)],
                    out_sh.at[pl.ds(sid * (NPAD // 16) + (nz - 1) * G, rem)])
    plsc.subcore_barrier()

    pltpu.async_copy(dst_hbm.at[wid, 0], db0, d0)
    pltpu.async_copy(dst_hbm.at[wid, 1], db1, d1)
    pltpu.async_copy(ex_hbm.at[wid, 0], xb0, x0)
    pltpu.async_copy(ex_hbm.at[wid, 1], xb1, x1)
    pltpu.async_copy(h_hbm.at[src_at(0)], rows0, g0)
    pltpu.async_copy(h_hbm.at[src_at(1)], rows1, g1)

    def do_slot(j, c, p, q):
        pltpu.make_async_copy(dst_hbm.at[wid, c], db[p], dsem[p]).wait()
        pltpu.make_async_copy(ex_hbm.at[wid, c], xb[p], xsem[p]).wait()
        pltpu.make_async_copy(h_hbm.at[src_at(c)], rows[p], gsem[p]).wait()

        cr = lax.shift_right_logical(c, 2)
        cc0 = jnp.bitwise_and(c, 3) * G
        for k in range(G // 16):
            i = pl.ds(cc0 + k * 16, 16)
            s16 = src2[cr, i]
            wbuf[pl.ds(k * 16, 16)] = xb[p][pl.ds(k * 16, 16)] * plsc.load_gather(
                den_v, [lax.shift_right_logical(s16, 7),
                        jnp.bitwise_and(s16, 127)])

        @pl.when(j > 0)
        def _():
            pltpu.make_async_copy(srows[q], out_sh.at[dx[q].at[0]],
                                  ssem[q]).wait()
        for k in range(G // 16):
            i = pl.ds(k * 16, 16)
            dx[q][0, i] = db[p][0, i]

        @plsc.parallel_loop(0, G, unroll=4)
        def scale(r):
            wb = plsc.load_gather(wbuf, [jnp.full((16,), r, jnp.int32)])
            for cc in range(8):
                i = pl.ds(cc * 16, 16)
                srows[q][r, i] = rows[p][r, i] * wb
        pltpu.async_copy(srows[q], out_sh.at[dx[q].at[0]], ssem[q], add=True)

        @pl.when(c + 2 < NCH)
        def _():
            pltpu.async_copy(dst_hbm.at[wid, c + 2], db[p], dsem[p])
            pltpu.async_copy(ex_hbm.at[wid, c + 2], xb[p], xsem[p])
            pltpu.async_copy(h_hbm.at[src_at(c + 2)], rows[p], gsem[p])

    def quad(j, _):
        for k in range(4):
            do_slot(j, 4 * j + k, k % 2, k)
        return 0
    lax.fori_loop(0, NCH // 4, quad, 0)
    for q in range(4):
        pltpu.make_async_copy(srows[q], out_sh.at[dx[q].at[0]], ssem[q]).wait()
    plsc.subcore_barrier()

    for j in range(nz - 1):
        r0 = sid * (NPAD // 16) + j * G
        pltpu.sync_copy(out_sh.at[pl.ds(r0, G)], out_hbm.at[cid, pl.ds(r0, G)])
    r0 = sid * (NPAD // 16) + (nz - 1) * G
    pltpu.sync_copy(out_sh.at[pl.ds(r0, rem)], out_hbm.at[cid, pl.ds(r0, rem)])


_sc_agg = pl.kernel(
    _sc_agg_body,
    out_type=jax.ShapeDtypeStruct((2, NPAD, D), jnp.float32),
    mesh=_mesh,
    compiler_params=pltpu.CompilerParams(needs_layout_passes=False),
    scratch_types=[
        pltpu.VMEM((NPAD // 128, 128), jnp.float32),
        pltpu.VMEM((NCH // 4, 4 * G), jnp.int32),
        pltpu.VMEM((1, G), jnp.int32),
        pltpu.VMEM((1, G), jnp.int32),
        pltpu.VMEM((G,), jnp.float32),
        pltpu.VMEM((G,), jnp.float32),
        pltpu.VMEM((1, G), jnp.int32),
        pltpu.VMEM((1, G), jnp.int32),
        pltpu.VMEM((1, G), jnp.int32),
        pltpu.VMEM((1, G), jnp.int32),
        pltpu.VMEM((G,), jnp.float32),
        pltpu.VMEM((G, D), jnp.float32),
        pltpu.VMEM((G, D), jnp.float32),
        pltpu.VMEM((G, D), jnp.float32),
        pltpu.VMEM((G, D), jnp.float32),
        pltpu.VMEM((G, D), jnp.float32),
        pltpu.VMEM((G, D), jnp.float32),
        pltpu.VMEM_SHARED((NPAD, D), jnp.float32),
        pltpu.SemaphoreType.DMA,
        pltpu.SemaphoreType.DMA,
        pltpu.SemaphoreType.DMA,
        pltpu.SemaphoreType.DMA,
        pltpu.SemaphoreType.DMA,
        pltpu.SemaphoreType.DMA,
        pltpu.SemaphoreType.DMA,
        pltpu.SemaphoreType.DMA,
        pltpu.SemaphoreType.DMA,
        pltpu.SemaphoreType.DMA,
    ])


def _attp(att):
    a = jnp.zeros((8, D), jnp.float32)
    return a.at[0].set(att[0, 0, :D]).at[1].set(att[0, 0, D:])


def kernel(x, edge_index, node_label, node_index,
           W1, att1, b1, W2, att2, b2, W3, att3, b3, outW, outb):
    del node_label
    x_pad = jnp.zeros((NPAD, D), jnp.float32).at[:N].set(x)
    loops = jnp.arange(N, dtype=jnp.int32)
    padi = jnp.full((EPAD - N_EDGES_TOTAL,), NPAD - 1, jnp.int32)
    src = jnp.concatenate([edge_index[0], loops, padi])
    dst = jnp.concatenate([edge_index[1], loops, padi])
    src3 = src.reshape(NW, NCH // 4, 4 * G)
    dst4 = dst.reshape(NW, NCH, 1, G)

    def layer(g_parts, W, att, first):
        attp = _attp(att)
        if first:
            h, aT = _tc_first(g_parts, W, attp)
        else:
            p, b_prev = g_parts
            h, aT = _tc_mid(p[0], p[1], b_prev.reshape(1, D), W, attp)
        ex, den = _sc_alpha(aT, src, dst)
        return _sc_agg(h, ex.reshape(NW, NCH, G), dst4, src3, den), h

    o1, _ = layer(x_pad, W1, att1, True)
    o2, _ = layer((o1, b1), W2, att2, False)
    o3, _ = layer((o2, b2), W3, att3, False)

    owp = jnp.zeros((D, D), jnp.float32).at[:3].set(outW)
    obp = jnp.zeros((1, D), jnp.float32).at[0, :3].set(outb)
    xe, lg, ypr = _tc_last(o3[0], o3[1], b3.reshape(1, D), owp, obp)

    x_embed = xe[:N]
    output = lg[:N, :3]
    ypred = ypr[0, :N]
    node_output = output[node_index]
    y_nodepred = ypred[node_index]
    return (x_embed, node_output, ypred, y_nodepred)


# speedup vs baseline: 22.1361x; 1.0204x over previous
import functools

import jax
import jax.numpy as jnp
from jax import lax
from jax.experimental import pallas as pl
from jax.experimental.pallas import tpu as pltpu
from jax.experimental.pallas import tpu_sc as plsc

N = 10000
NPAD = 10240
D = 128
N_EDGES_TOTAL = 330000
NW = 32
G = 32
NCH = 324
EW = NCH * G
EPAD = NW * EW
ROWB = 1024
NBLK = NPAD // ROWB


def _act(p0, p1, b):
    g = jnp.maximum(p0 + p1 + b, 0.0)
    return jnp.maximum(g, 0.3 * g)


def _tc_first_body(x_ref, w_ref, attp_ref, h_ref, at_ref):
    g = x_ref[...]
    h = lax.dot_general(g, w_ref[...], (((1,), (1,)), ((), ())),
                        preferred_element_type=jnp.float32)
    h_ref[...] = h
    at_ref[...] = lax.dot_general(attp_ref[...], h, (((1,), (1,)), ((), ())),
                                  preferred_element_type=jnp.float32)


def _tc_mid_body(p0_ref, p1_ref, b_ref, w_ref, attp_ref, h_ref, at_ref):
    g = _act(p0_ref[...], p1_ref[...], b_ref[...])
    h = lax.dot_general(g, w_ref[...], (((1,), (1,)), ((), ())),
                        preferred_element_type=jnp.float32)
    h_ref[...] = h
    at_ref[...] = lax.dot_general(attp_ref[...], h, (((1,), (1,)), ((), ())),
                                  preferred_element_type=jnp.float32)


def _tc_last_body(p0_ref, p1_ref, b_ref, ow_ref, ob_ref, xe_ref, lg_ref, yp_ref):
    xe = _act(p0_ref[...], p1_ref[...], b_ref[...])
    xe_ref[...] = xe
    lg = lax.dot_general(xe, ow_ref[...], (((1,), (1,)), ((), ())),
                         preferred_element_type=jnp.float32) + ob_ref[...]
    lg_ref[...] = lg
    l0 = lg[:, 0]
    l1 = lg[:, 1]
    l2 = lg[:, 2]
    yp = jnp.where((l0 >= l1) & (l0 >= l2), 0, jnp.where(l1 >= l2, 1, 2))
    yp_ref[...] = jnp.zeros(yp_ref.shape, jnp.int32)
    yp_ref[0, :] = yp.astype(jnp.int32)


_row_spec = pl.BlockSpec((ROWB, D), lambda i: (i, 0))
_full_spec = pl.BlockSpec((D, D), lambda i: (0, 0))
_attp_spec = pl.BlockSpec((8, D), lambda i: (0, 0))
_b_spec = pl.BlockSpec((1, D), lambda i: (0, 0))
_at_spec = pl.BlockSpec((8, ROWB), lambda i: (0, i))

_h_at_out = (jax.ShapeDtypeStruct((NPAD, D), jnp.float32),
             jax.ShapeDtypeStruct((8, NPAD), jnp.float32))

_tc_first = pl.pallas_call(
    _tc_first_body, grid=(NBLK,),
    in_specs=[_row_spec, _full_spec, _attp_spec],
    out_specs=(_row_spec, _at_spec),
    out_shape=_h_at_out)

_tc_mid = pl.pallas_call(
    _tc_mid_body, grid=(NBLK,),
    in_specs=[_row_spec, _row_spec, _b_spec, _full_spec, _attp_spec],
    out_specs=(_row_spec, _at_spec),
    out_shape=_h_at_out)

_tc_last = pl.pallas_call(
    _tc_last_body, grid=(NBLK,),
    in_specs=[_row_spec, _row_spec, _b_spec, _full_spec, _b_spec],
    out_specs=(_row_spec, _row_spec, pl.BlockSpec((8, ROWB), lambda i: (0, i))),
    out_shape=(jax.ShapeDtypeStruct((NPAD, D), jnp.float32),
               jax.ShapeDtypeStruct((NPAD, D), jnp.float32),
               jax.ShapeDtypeStruct((8, NPAD), jnp.int32)))


_mesh = plsc.VectorSubcoreMesh(core_axis_name="c", subcore_axis_name="s")


def _sc_alpha_body(at_hbm, src_hbm, dst_hbm, ex_hbm, den_hbm,
                   ai_v, aj_v, den_v, idi_v, src_v, dst_v, ex_v, den_sh, sem):
    cid = lax.axis_index("c")
    sid = lax.axis_index("s")
    wid = sid * 2 + cid
    base = wid * EW
    pltpu.async_copy(at_hbm.at[0], ai_v, sem)
    pltpu.async_copy(at_hbm.at[1], aj_v, sem)
    pltpu.async_copy(src_hbm.at[pl.ds(base, EW)], src_v, sem)
    pltpu.async_copy(dst_hbm.at[pl.ds(base, EW)], dst_v, sem)
    pltpu.make_async_copy(at_hbm.at[0], ai_v, sem).wait()
    pltpu.make_async_copy(at_hbm.at[1], aj_v, sem).wait()
    pltpu.make_async_copy(src_hbm.at[pl.ds(base, EW)], src_v, sem).wait()
    pltpu.make_async_copy(dst_hbm.at[pl.ds(base, EW)], dst_v, sem).wait()

    def mx(k, acc):
        return jnp.maximum(acc, ai_v[pl.ds(k * 16, 16)])
    acc = lax.fori_loop(0, NPAD // 16, mx, ai_v[pl.ds(0, 16)])
    lanes = lax.iota(jnp.int32, 16)
    for sh in (8, 4, 2, 1):
        ex_v[pl.ds(0, 16)] = acc
        acc = jnp.maximum(acc, plsc.load_gather(ex_v, [lanes ^ sh]))
    amax = acc

    for m in range(5):
        idi_v[0, pl.ds(m * 16, 16)] = lanes + m * 16

    zero16 = jnp.zeros((16,), jnp.float32)

    def zz(k, _):
        den_v[k // 8, pl.ds((k % 8) * 16, 16)] = zero16
        return 0
    lax.fori_loop(0, NPAD // 16, zz, 0)

    @pl.when(sid == 0)
    def _():
        pltpu.sync_copy(den_v, den_sh)
    plsc.subcore_barrier()

    def step(k, _):
        s16 = src_v[pl.ds(k * 16, 16)]
        d16 = dst_v[pl.ds(k * 16, 16)]
        ai = plsc.load_gather(ai_v, [d16])
        aj = plsc.load_gather(aj_v, [s16])
        ssum = ai + aj
        z = jnp.maximum(ssum, 0.2 * ssum)
        t = amax + aj
        c = jnp.maximum(t, 0.2 * t)
        ex = jnp.exp(z - c)
        ex_v[pl.ds(k * 16, 16)] = ex
        plsc.addupdate_scatter(
            den_v, [lax.shift_right_logical(s16, 7), jnp.bitwise_and(s16, 127)], ex)
        return 0
    lax.fori_loop(0, EW // 16, step, 0)

    pltpu.sync_copy(ex_v, ex_hbm.at[pl.ds(base, EW)])
    pltpu.sync_copy(den_v, den_sh.at[idi_v.at[0]], add=True)
    plsc.subcore_barrier()

    @pl.when(sid < 10)
    def _():
        pltpu.sync_copy(den_sh.at[pl.ds(sid * 8, 8)],
                        den_hbm.at[cid, pl.ds(sid * 8, 8)])


_sc_alpha = pl.kernel(
    _sc_alpha_body,
    out_type=(jax.ShapeDtypeStruct((EPAD,), jnp.float32),
              jax.ShapeDtypeStruct((2, NPAD // 128, 128), jnp.float32)),
    mesh=_mesh,
    compiler_params=pltpu.CompilerParams(needs_layout_passes=False),
    scratch_types=[
        pltpu.VMEM((NPAD,), jnp.float32),
        pltpu.VMEM((NPAD,), jnp.float32),
        pltpu.VMEM((NPAD // 128, 128), jnp.float32),
        pltpu.VMEM((1, NPAD // 128), jnp.int32),
        pltpu.VMEM((EW,), jnp.int32),
        pltpu.VMEM((EW,), jnp.int32),
        pltpu.VMEM((EW,), jnp.float32),
        pltpu.VMEM_SHARED((NPAD // 128, 128), jnp.float32),
        pltpu.SemaphoreType.DMA,
    ])


def _sc_agg_body(h_hbm, ex_hbm, dst_hbm, src_hbm, den_hbm, out_hbm,
                 den_v, src2, db0, db1, xb0, xb1, dx0, dx1, dx2, dx3, wbuf,
                 rows0, rows1, srows0, srows1, srows2, srows3, out_sh,
                 g0, g1, s0, s1, s2, s3, d0, d1, x0, x1):
    cid = lax.axis_index("c")
    sid = lax.axis_index("s")
    wid = sid * 2 + cid
    rows = (rows0, rows1)
    srows = (srows0, srows1, srows2, srows3)
    db = (db0, db1)
    xb = (xb0, xb1)
    dx = (dx0, dx1, dx2, dx3)
    gsem = (g0, g1)
    ssem = (s0, s1, s2, s3)
    dsem = (d0, d1)
    xsem = (x0, x1)

    pltpu.sync_copy(src_hbm.at[wid], src2)

    def src_at(c):
        return src2.at[lax.shift_right_logical(c, 2),
                       pl.ds(jnp.bitwise_and(c, 3) * G, G)]

    pltpu.sync_copy(den_hbm.at[0], den_v)
    for buf, off, nn in ((rows0, 0, 32), (rows1, 32, 32), (srows0, 64, 16)):
        pltpu.sync_copy(den_hbm.at[1, pl.ds(off, nn)], buf.at[pl.ds(0, nn)])

        def rcp(k, _, buf=buf, off=off):
            r, i = k // 8, pl.ds((k % 8) * 16, 16)
            den_v[off + r, i] = 1.0 / (den_v[off + r, i] + buf[r, i] + 1e-16)
            return 0
        lax.fori_loop(0, nn * 8, rcp, 0)

    zero16 = jnp.zeros((16,), jnp.float32)

    def zrow(k, _):
        srows0[k // 8, pl.ds((k % 8) * 16, 16)] = zero16
        return 0
    lax.fori_loop(0, G * 8, zrow, 0)
    nz = NPAD // 16 // G
    for j in range(nz):
        pltpu.async_copy(srows0, out_sh.at[pl.ds(sid * (NPAD // 16) + j * G, G)],
                         g0)
    for j in range(nz):
        pltpu.make_async_copy(
            srows0, out_sh.at[pl.ds(sid * (NPAD // 16) + j * G, G)], g0).wait()
    plsc.subcore_barrier()

    pltpu.async_copy(dst_hbm.at[wid, 0], db0, d0)
    pltpu.async_copy(dst_hbm.at[wid, 1], db1, d1)
    pltpu.async_copy(ex_hbm.at[wid, 0], xb0, x0)
    pltpu.async_copy(ex_hbm.at[wid, 1], xb1, x1)
    pltpu.async_copy(h_hbm.at[src_at(0)], rows0, g0)
    pltpu.async_copy(h_hbm.at[src_at(1)], rows1, g1)

    def do_slot(j, c, p, q):
        pltpu.make_async_copy(dst_hbm.at[wid, c], db[p], dsem[p]).wait()
        pltpu.make_async_copy(ex_hbm.at[wid, c], xb[p], xsem[p]).wait()
        pltpu.make_async_copy(h_hbm.at[src_at(c)], rows[p], gsem[p]).wait()

        cr = lax.shift_right_logical(c, 2)
        cc0 = jnp.bitwise_and(c, 3) * G
        for k in range(G // 16):
            i = pl.ds(cc0 + k * 16, 16)
            s16 = src2[cr, i]
            wbuf[pl.ds(k * 16, 16)] = xb[p][pl.ds(k * 16, 16)] * plsc.load_gather(
                den_v, [lax.shift_right_logical(s16, 7),
                        jnp.bitwise_and(s16, 127)])

        @pl.when(j > 0)
        def _():
            pltpu.make_async_copy(srows[q], out_sh.at[dx[q].at[0]],
                                  ssem[q]).wait()
        for k in range(G // 16):
            i = pl.ds(k * 16, 16)
            dx[q][0, i] = db[p][0, i]

        @plsc.parallel_loop(0, G, unroll=4)
        def scale(r):
            wb = plsc.load_gather(wbuf, [jnp.full((16,), r, jnp.int32)])
            for cc in range(8):
                i = pl.ds(cc * 16, 16)
                srows[q][r, i] = rows[p][r, i] * wb
        pltpu.async_copy(srows[q], out_sh.at[dx[q].at[0]], ssem[q], add=True)

        @pl.when(c + 2 < NCH)
        def _():
            pltpu.async_copy(dst_hbm.at[wid, c + 2], db[p], dsem[p])
            pltpu.async_copy(ex_hbm.at[wid, c + 2], xb[p], xsem[p])
            pltpu.async_copy(h_hbm.at[src_at(c + 2)], rows[p], gsem[p])

    def quad(j, _):
        for k in range(4):
            do_slot(j, 4 * j + k, k % 2, k)
        return 0
    lax.fori_loop(0, NCH // 4, quad, 0)
    for q in range(4):
        pltpu.make_async_copy(srows[q], out_sh.at[dx[q].at[0]], ssem[q]).wait()
    plsc.subcore_barrier()

    for j in range(nz):
        r0 = sid * (NPAD // 16) + j * G
        pltpu.async_copy(out_sh.at[pl.ds(r0, G)], out_hbm.at[cid, pl.ds(r0, G)],
                         g0)
    for j in range(nz):
        r0 = sid * (NPAD // 16) + j * G
        pltpu.make_async_copy(out_sh.at[pl.ds(r0, G)],
                              out_hbm.at[cid, pl.ds(r0, G)], g0).wait()


_sc_agg = pl.kernel(
    _sc_agg_body,
    out_type=jax.ShapeDtypeStruct((2, NPAD, D), jnp.float32),
    mesh=_mesh,
    compiler_params=pltpu.CompilerParams(needs_layout_passes=False),
    scratch_types=[
        pltpu.VMEM((NPAD // 128, 128), jnp.float32),
        pltpu.VMEM((NCH // 4, 4 * G), jnp.int32),
        pltpu.VMEM((1, G), jnp.int32),
        pltpu.VMEM((1, G), jnp.int32),
        pltpu.VMEM((G,), jnp.float32),
        pltpu.VMEM((G,), jnp.float32),
        pltpu.VMEM((1, G), jnp.int32),
        pltpu.VMEM((1, G), jnp.int32),
        pltpu.VMEM((1, G), jnp.int32),
        pltpu.VMEM((1, G), jnp.int32),
        pltpu.VMEM((G,), jnp.float32),
        pltpu.VMEM((G, D), jnp.float32),
        pltpu.VMEM((G, D), jnp.float32),
        pltpu.VMEM((G, D), jnp.float32),
        pltpu.VMEM((G, D), jnp.float32),
        pltpu.VMEM((G, D), jnp.float32),
        pltpu.VMEM((G, D), jnp.float32),
        pltpu.VMEM_SHARED((NPAD, D), jnp.float32),
        pltpu.SemaphoreType.DMA,
        pltpu.SemaphoreType.DMA,
        pltpu.SemaphoreType.DMA,
        pltpu.SemaphoreType.DMA,
        pltpu.SemaphoreType.DMA,
        pltpu.SemaphoreType.DMA,
        pltpu.SemaphoreType.DMA,
        pltpu.SemaphoreType.DMA,
        pltpu.SemaphoreType.DMA,
        pltpu.SemaphoreType.DMA,
    ])


def _attp(att):
    a = jnp.zeros((8, D), jnp.float32)
    return a.at[0].set(att[0, 0, :D]).at[1].set(att[0, 0, D:])


def kernel(x, edge_index, node_label, node_index,
           W1, att1, b1, W2, att2, b2, W3, att3, b3, outW, outb):
    del node_label
    x_pad = jnp.zeros((NPAD, D), jnp.float32).at[:N].set(x)
    loops = jnp.arange(N, dtype=jnp.int32)
    padi = jnp.full((EPAD - N_EDGES_TOTAL,), NPAD - 1, jnp.int32)
    src = jnp.concatenate([edge_index[0], loops, padi])
    dst = jnp.concatenate([edge_index[1], loops, padi])
    src3 = src.reshape(NW, NCH // 4, 4 * G)
    dst4 = dst.reshape(NW, NCH, 1, G)

    def layer(g_parts, W, att, first):
        attp = _attp(att)
        if first:
            h, aT = _tc_first(g_parts, W, attp)
        else:
            p, b_prev = g_parts
            h, aT = _tc_mid(p[0], p[1], b_prev.reshape(1, D), W, attp)
        ex, den = _sc_alpha(aT, src, dst)
        return _sc_agg(h, ex.reshape(NW, NCH, G), dst4, src3, den), h

    o1, _ = layer(x_pad, W1, att1, True)
    o2, _ = layer((o1, b1), W2, att2, False)
    o3, _ = layer((o2, b2), W3, att3, False)

    owp = jnp.zeros((D, D), jnp.float32).at[:3].set(outW)
    obp = jnp.zeros((1, D), jnp.float32).at[0, :3].set(outb)
    xe, lg, ypr = _tc_last(o3[0], o3[1], b3.reshape(1, D), owp, obp)

    x_embed = xe[:N]
    output = lg[:N, :3]
    ypred = ypr[0, :N]
    node_output = output[node_index]
    y_nodepred = ypred[node_index]
    return (x_embed, node_output, ypred, y_nodepred)


# speedup vs baseline: 23.2087x; 1.0485x over previous
import functools

import jax
import jax.numpy as jnp
from jax import lax
from jax.experimental import pallas as pl
from jax.experimental.pallas import tpu as pltpu
from jax.experimental.pallas import tpu_sc as plsc

N = 10000
NPAD = 10240
D = 128
N_EDGES_TOTAL = 330000
NW = 32
G = 32
NCH = 324
EW = NCH * G
EPAD = NW * EW
ROWB = 1024
NBLK = NPAD // ROWB


def _act(p0, p1, b):
    g = jnp.maximum(p0 + p1 + b, 0.0)
    return jnp.maximum(g, 0.3 * g)


def _tc_first_body(x_ref, w_ref, attp_ref, h_ref, at_ref):
    g = x_ref[...]
    h = lax.dot_general(g, w_ref[...], (((1,), (1,)), ((), ())),
                        preferred_element_type=jnp.float32)
    h_ref[...] = h
    at_ref[...] = lax.dot_general(attp_ref[...], h, (((1,), (1,)), ((), ())),
                                  preferred_element_type=jnp.float32)


def _tc_mid_body(p0_ref, p1_ref, b_ref, w_ref, attp_ref, h_ref, at_ref):
    g = _act(p0_ref[...], p1_ref[...], b_ref[...])
    h = lax.dot_general(g, w_ref[...], (((1,), (1,)), ((), ())),
                        preferred_element_type=jnp.float32)
    h_ref[...] = h
    at_ref[...] = lax.dot_general(attp_ref[...], h, (((1,), (1,)), ((), ())),
                                  preferred_element_type=jnp.float32)


def _tc_last_body(p0_ref, p1_ref, b_ref, ow_ref, ob_ref, xe_ref, lg_ref, yp_ref):
    xe = _act(p0_ref[...], p1_ref[...], b_ref[...])
    xe_ref[...] = xe
    lg = lax.dot_general(xe, ow_ref[...], (((1,), (1,)), ((), ())),
                         preferred_element_type=jnp.float32) + ob_ref[...]
    lg_ref[...] = lg
    l0 = lg[:, 0]
    l1 = lg[:, 1]
    l2 = lg[:, 2]
    yp = jnp.where((l0 >= l1) & (l0 >= l2), 0, jnp.where(l1 >= l2, 1, 2))
    yp_ref[...] = jnp.zeros(yp_ref.shape, jnp.int32)
    yp_ref[0, :] = yp.astype(jnp.int32)


_row_spec = pl.BlockSpec((ROWB, D), lambda i: (i, 0))
_full_spec = pl.BlockSpec((D, D), lambda i: (0, 0))
_attp_spec = pl.BlockSpec((8, D), lambda i: (0, 0))
_b_spec = pl.BlockSpec((1, D), lambda i: (0, 0))
_at_spec = pl.BlockSpec((8, ROWB), lambda i: (0, i))

_h_at_out = (jax.ShapeDtypeStruct((NPAD, D), jnp.float32),
             jax.ShapeDtypeStruct((8, NPAD), jnp.float32))

_tc_first = pl.pallas_call(
    _tc_first_body, grid=(NBLK,),
    in_specs=[_row_spec, _full_spec, _attp_spec],
    out_specs=(_row_spec, _at_spec),
    out_shape=_h_at_out)

_tc_mid = pl.pallas_call(
    _tc_mid_body, grid=(NBLK,),
    in_specs=[_row_spec, _row_spec, _b_spec, _full_spec, _attp_spec],
    out_specs=(_row_spec, _at_spec),
    out_shape=_h_at_out)

_tc_last = pl.pallas_call(
    _tc_last_body, grid=(NBLK,),
    in_specs=[_row_spec, _row_spec, _b_spec, _full_spec, _b_spec],
    out_specs=(_row_spec, _row_spec, pl.BlockSpec((8, ROWB), lambda i: (0, i))),
    out_shape=(jax.ShapeDtypeStruct((NPAD, D), jnp.float32),
               jax.ShapeDtypeStruct((NPAD, D), jnp.float32),
               jax.ShapeDtypeStruct((8, NPAD), jnp.int32)))


_mesh = plsc.VectorSubcoreMesh(core_axis_name="c", subcore_axis_name="s")


def _sc_alpha_body(at_hbm, src_hbm, dst_hbm, ex_hbm, den_hbm,
                   ai_v, aj_v, den_v, idi_v, src_v, dst_v, ex_v, den_sh, sem):
    cid = lax.axis_index("c")
    sid = lax.axis_index("s")
    wid = sid * 2 + cid
    base = wid * EW
    pltpu.async_copy(at_hbm.at[0], ai_v, sem)
    pltpu.async_copy(at_hbm.at[1], aj_v, sem)
    pltpu.async_copy(src_hbm.at[pl.ds(base, EW)], src_v, sem)
    pltpu.async_copy(dst_hbm.at[pl.ds(base, EW)], dst_v, sem)
    pltpu.make_async_copy(at_hbm.at[0], ai_v, sem).wait()
    pltpu.make_async_copy(at_hbm.at[1], aj_v, sem).wait()
    pltpu.make_async_copy(src_hbm.at[pl.ds(base, EW)], src_v, sem).wait()
    pltpu.make_async_copy(dst_hbm.at[pl.ds(base, EW)], dst_v, sem).wait()

    def mx(k, acc):
        return jnp.maximum(acc, ai_v[pl.ds(k * 16, 16)])
    acc = lax.fori_loop(0, NPAD // 16, mx, ai_v[pl.ds(0, 16)])
    lanes = lax.iota(jnp.int32, 16)
    for sh in (8, 4, 2, 1):
        ex_v[pl.ds(0, 16)] = acc
        acc = jnp.maximum(acc, plsc.load_gather(ex_v, [lanes ^ sh]))
    amax = acc

    for m in range(5):
        idi_v[0, pl.ds(m * 16, 16)] = lanes + m * 16

    zero16 = jnp.zeros((16,), jnp.float32)

    def zz(k, _):
        den_v[k // 8, pl.ds((k % 8) * 16, 16)] = zero16
        return 0
    lax.fori_loop(0, NPAD // 16, zz, 0)

    @pl.when(sid == 0)
    def _():
        pltpu.sync_copy(den_v, den_sh)
    plsc.subcore_barrier()

    def step(k, _):
        s16 = src_v[pl.ds(k * 16, 16)]
        d16 = dst_v[pl.ds(k * 16, 16)]
        ai = plsc.load_gather(ai_v, [d16])
        aj = plsc.load_gather(aj_v, [s16])
        ssum = ai + aj
        z = jnp.maximum(ssum, 0.2 * ssum)
        t = amax + aj
        c = jnp.maximum(t, 0.2 * t)
        ex = jnp.exp(z - c)
        ex_v[pl.ds(k * 16, 16)] = ex
        plsc.addupdate_scatter(
            den_v, [lax.shift_right_logical(s16, 7), jnp.bitwise_and(s16, 127)], ex)
        return 0
    lax.fori_loop(0, EW // 16, step, 0)

    pltpu.sync_copy(ex_v, ex_hbm.at[pl.ds(base, EW)])
    pltpu.sync_copy(den_v, den_sh.at[idi_v.at[0]], add=True)
    plsc.subcore_barrier()

    @pl.when(sid < 10)
    def _():
        pltpu.sync_copy(den_sh.at[pl.ds(sid * 8, 8)],
                        den_hbm.at[cid, pl.ds(sid * 8, 8)])


_sc_alpha = pl.kernel(
    _sc_alpha_body,
    out_type=(jax.ShapeDtypeStruct((EPAD,), jnp.float32),
              jax.ShapeDtypeStruct((2, NPAD // 128, 128), jnp.float32)),
    mesh=_mesh,
    compiler_params=pltpu.CompilerParams(needs_layout_passes=False),
    scratch_types=[
        pltpu.VMEM((NPAD,), jnp.float32),
        pltpu.VMEM((NPAD,), jnp.float32),
        pltpu.VMEM((NPAD // 128, 128), jnp.float32),
        pltpu.VMEM((1, NPAD // 128), jnp.int32),
        pltpu.VMEM((EW,), jnp.int32),
        pltpu.VMEM((EW,), jnp.int32),
        pltpu.VMEM((EW,), jnp.float32),
        pltpu.VMEM_SHARED((NPAD // 128, 128), jnp.float32),
        pltpu.SemaphoreType.DMA,
    ])


def _sc_agg_body(h_hbm, ex_hbm, dst_hbm, src_hbm, den_hbm, out_hbm,
                 den_v, src2, db0, db1, xb0, xb1, dx0, dx1, dx2, wbuf,
                 rows0, rows1, rows2, srows0, srows1, srows2, out_sh,
                 g0, g1, g2, s0, s1, s2, d0, d1, x0, x1):
    cid = lax.axis_index("c")
    sid = lax.axis_index("s")
    wid = sid * 2 + cid
    rows = (rows0, rows1, rows2)
    srows = (srows0, srows1, srows2)
    db = (db0, db1)
    xb = (xb0, xb1)
    dx = (dx0, dx1, dx2)
    gsem = (g0, g1, g2)
    ssem = (s0, s1, s2)
    dsem = (d0, d1)
    xsem = (x0, x1)

    pltpu.sync_copy(src_hbm.at[wid], src2)

    def src_at(c):
        return src2.at[lax.shift_right_logical(c, 2),
                       pl.ds(jnp.bitwise_and(c, 3) * G, G)]

    pltpu.sync_copy(den_hbm.at[0], den_v)
    for buf, off, nn in ((rows0, 0, 32), (rows1, 32, 32), (rows2, 64, 16)):
        pltpu.sync_copy(den_hbm.at[1, pl.ds(off, nn)], buf.at[pl.ds(0, nn)])

        def rcp(k, _, buf=buf, off=off):
            r, i = k // 8, pl.ds((k % 8) * 16, 16)
            den_v[off + r, i] = 1.0 / (den_v[off + r, i] + buf[r, i] + 1e-16)
            return 0
        lax.fori_loop(0, nn * 8, rcp, 0)

    zero16 = jnp.zeros((16,), jnp.float32)

    def zrow(k, _):
        srows0[k // 8, pl.ds((k % 8) * 16, 16)] = zero16
        return 0
    lax.fori_loop(0, G * 8, zrow, 0)
    nz = NPAD // 16 // G
    for j in range(nz):
        pltpu.async_copy(srows0, out_sh.at[pl.ds(sid * (NPAD // 16) + j * G, G)],
                         g0)
    for j in range(nz):
        pltpu.make_async_copy(
            srows0, out_sh.at[pl.ds(sid * (NPAD // 16) + j * G, G)], g0).wait()
    plsc.subcore_barrier()

    pltpu.async_copy(dst_hbm.at[wid, 0], db0, d0)
    pltpu.async_copy(dst_hbm.at[wid, 1], db1, d1)
    pltpu.async_copy(ex_hbm.at[wid, 0], xb0, x0)
    pltpu.async_copy(ex_hbm.at[wid, 1], xb1, x1)
    pltpu.async_copy(h_hbm.at[src_at(0)], rows0, g0)
    pltpu.async_copy(h_hbm.at[src_at(1)], rows1, g1)
    pltpu.async_copy(h_hbm.at[src_at(2)], rows2, g2)

    def do_slot(j, c, k):
        p = k % 2
        q = k % 3
        pltpu.make_async_copy(dst_hbm.at[wid, c], db[p], dsem[p]).wait()
        pltpu.make_async_copy(ex_hbm.at[wid, c], xb[p], xsem[p]).wait()
        pltpu.make_async_copy(h_hbm.at[src_at(c)], rows[q], gsem[q]).wait()

        cr = lax.shift_right_logical(c, 2)
        cc0 = jnp.bitwise_and(c, 3) * G
        for m in range(G // 16):
            i = pl.ds(cc0 + m * 16, 16)
            s16 = src2[cr, i]
            wbuf[pl.ds(m * 16, 16)] = xb[p][pl.ds(m * 16, 16)] * plsc.load_gather(
                den_v, [lax.shift_right_logical(s16, 7),
                        jnp.bitwise_and(s16, 127)])

        @pl.when(c >= 3)
        def _():
            pltpu.make_async_copy(srows[q], out_sh.at[dx[q].at[0]],
                                  ssem[q]).wait()
        for m in range(G // 16):
            i = pl.ds(m * 16, 16)
            dx[q][0, i] = db[p][0, i]

        @plsc.parallel_loop(0, G, unroll=4)
        def scale(r):
            wb = plsc.load_gather(wbuf, [jnp.full((16,), r, jnp.int32)])
            for cc in range(8):
                i = pl.ds(cc * 16, 16)
                srows[q][r, i] = rows[q][r, i] * wb
        pltpu.async_copy(srows[q], out_sh.at[dx[q].at[0]], ssem[q], add=True)

        @pl.when(c + 2 < NCH)
        def _():
            pltpu.async_copy(dst_hbm.at[wid, c + 2], db[p], dsem[p])
            pltpu.async_copy(ex_hbm.at[wid, c + 2], xb[p], xsem[p])

        @pl.when(c + 3 < NCH)
        def _():
            pltpu.async_copy(h_hbm.at[src_at(c + 3)], rows[q], gsem[q])

    def dodeca(j, _):
        for k in range(12):
            do_slot(j, 12 * j + k, k)
        return 0
    lax.fori_loop(0, NCH // 12, dodeca, 0)
    for c in range(NCH - 3, NCH):
        q = c % 3
        pltpu.make_async_copy(srows[q], out_sh.at[dx[q].at[0]], ssem[q]).wait()
    plsc.subcore_barrier()

    for j in range(nz):
        r0 = sid * (NPAD // 16) + j * G
        pltpu.async_copy(out_sh.at[pl.ds(r0, G)], out_hbm.at[cid, pl.ds(r0, G)],
                         g0)
    for j in range(nz):
        r0 = sid * (NPAD // 16) + j * G
        pltpu.make_async_copy(out_sh.at[pl.ds(r0, G)],
                              out_hbm.at[cid, pl.ds(r0, G)], g0).wait()


_sc_agg = pl.kernel(
    _sc_agg_body,
    out_type=jax.ShapeDtypeStruct((2, NPAD, D), jnp.float32),
    mesh=_mesh,
    compiler_params=pltpu.CompilerParams(needs_layout_passes=False),
    scratch_types=[
        pltpu.VMEM((NPAD // 128, 128), jnp.float32),
        pltpu.VMEM((NCH // 4, 4 * G), jnp.int32),
        pltpu.VMEM((1, G), jnp.int32),
        pltpu.VMEM((1, G), jnp.int32),
        pltpu.VMEM((G,), jnp.float32),
        pltpu.VMEM((G,), jnp.float32),
        pltpu.VMEM((1, G), jnp.int32),
        pltpu.VMEM((1, G), jnp.int32),
        pltpu.VMEM((1, G), jnp.int32),
        pltpu.VMEM((G,), jnp.float32),
        pltpu.VMEM((G, D), jnp.float32),
        pltpu.VMEM((G, D), jnp.float32),
        pltpu.VMEM((G, D), jnp.float32),
        pltpu.VMEM((G, D), jnp.float32),
        pltpu.VMEM((G, D), jnp.float32),
        pltpu.VMEM((G, D), jnp.float32),
        pltpu.VMEM_SHARED((NPAD, D), jnp.float32),
        pltpu.SemaphoreType.DMA,
        pltpu.SemaphoreType.DMA,
        pltpu.SemaphoreType.DMA,
        pltpu.SemaphoreType.DMA,
        pltpu.SemaphoreType.DMA,
        pltpu.SemaphoreType.DMA,
        pltpu.SemaphoreType.DMA,
        pltpu.SemaphoreType.DMA,
        pltpu.SemaphoreType.DMA,
        pltpu.SemaphoreType.DMA,
    ])


def _attp(att):
    a = jnp.zeros((8, D), jnp.float32)
    return a.at[0].set(att[0, 0, :D]).at[1].set(att[0, 0, D:])


def kernel(x, edge_index, node_label, node_index,
           W1, att1, b1, W2, att2, b2, W3, att3, b3, outW, outb):
    del node_label
    x_pad = jnp.zeros((NPAD, D), jnp.float32).at[:N].set(x)
    loops = jnp.arange(N, dtype=jnp.int32)
    padi = jnp.full((EPAD - N_EDGES_TOTAL,), NPAD - 1, jnp.int32)
    src = jnp.concatenate([edge_index[0], loops, padi])
    dst = jnp.concatenate([edge_index[1], loops, padi])
    src3 = src.reshape(NW, NCH // 4, 4 * G)
    dst4 = dst.reshape(NW, NCH, 1, G)

    def layer(g_parts, W, att, first):
        attp = _attp(att)
        if first:
            h, aT = _tc_first(g_parts, W, attp)
        else:
            p, b_prev = g_parts
            h, aT = _tc_mid(p[0], p[1], b_prev.reshape(1, D), W, attp)
        ex, den = _sc_alpha(aT, src, dst)
        return _sc_agg(h, ex.reshape(NW, NCH, G), dst4, src3, den), h

    o1, _ = layer(x_pad, W1, att1, True)
    o2, _ = layer((o1, b1), W2, att2, False)
    o3, _ = layer((o2, b2), W3, att3, False)

    owp = jnp.zeros((D, D), jnp.float32).at[:3].set(outW)
    obp = jnp.zeros((1, D), jnp.float32).at[0, :3].set(outb)
    xe, lg, ypr = _tc_last(o3[0], o3[1], b3.reshape(1, D), owp, obp)

    x_embed = xe[:N]
    output = lg[:N, :3]
    ypred = ypr[0, :N]
    node_output = output[node_index]
    y_nodepred = ypred[node_index]
    return (x_embed, node_output, ypred, y_nodepred)


# speedup vs baseline: 23.6016x; 1.0169x over previous
import functools

import jax
import jax.numpy as jnp
from jax import lax
from jax.experimental import pallas as pl
from jax.experimental.pallas import tpu as pltpu
from jax.experimental.pallas import tpu_sc as plsc

N = 10000
NPAD = 10240
D = 128
N_EDGES_TOTAL = 330000
NW = 32
G = 32
NCH = 324
EW = NCH * G
EPAD = NW * EW
ROWB = 1024
NBLK = NPAD // ROWB


def _act(p0, p1, b):
    g = jnp.maximum(p0 + p1 + b, 0.0)
    return jnp.maximum(g, 0.3 * g)


def _tc_first_body(x_ref, w_ref, attp_ref, h_ref, at_ref):
    g = x_ref[...]
    h = lax.dot_general(g, w_ref[...], (((1,), (1,)), ((), ())),
                        preferred_element_type=jnp.float32)
    h_ref[...] = h
    at_ref[...] = lax.dot_general(attp_ref[...], h, (((1,), (1,)), ((), ())),
                                  preferred_element_type=jnp.float32)


def _tc_mid_body(p0_ref, p1_ref, b_ref, w_ref, attp_ref, h_ref, at_ref):
    g = _act(p0_ref[...], p1_ref[...], b_ref[...])
    h = lax.dot_general(g, w_ref[...], (((1,), (1,)), ((), ())),
                        preferred_element_type=jnp.float32)
    h_ref[...] = h
    at_ref[...] = lax.dot_general(attp_ref[...], h, (((1,), (1,)), ((), ())),
                                  preferred_element_type=jnp.float32)


def _tc_last_body(p0_ref, p1_ref, b_ref, ow_ref, ob_ref, xe_ref, lg_ref, yp_ref):
    xe = _act(p0_ref[...], p1_ref[...], b_ref[...])
    xe_ref[...] = xe
    lg = lax.dot_general(xe, ow_ref[...], (((1,), (1,)), ((), ())),
                         preferred_element_type=jnp.float32) + ob_ref[...]
    lg_ref[...] = lg
    l0 = lg[:, 0]
    l1 = lg[:, 1]
    l2 = lg[:, 2]
    yp = jnp.where((l0 >= l1) & (l0 >= l2), 0, jnp.where(l1 >= l2, 1, 2))
    yp_ref[...] = jnp.zeros(yp_ref.shape, jnp.int32)
    yp_ref[0, :] = yp.astype(jnp.int32)


_row_spec = pl.BlockSpec((ROWB, D), lambda i: (i, 0))
_full_spec = pl.BlockSpec((D, D), lambda i: (0, 0))
_attp_spec = pl.BlockSpec((8, D), lambda i: (0, 0))
_b_spec = pl.BlockSpec((1, D), lambda i: (0, 0))
_at_spec = pl.BlockSpec((8, ROWB), lambda i: (0, i))

_h_at_out = (jax.ShapeDtypeStruct((NPAD, D), jnp.float32),
             jax.ShapeDtypeStruct((8, NPAD), jnp.float32))

_tc_first = pl.pallas_call(
    _tc_first_body, grid=(NBLK,),
    in_specs=[_row_spec, _full_spec, _attp_spec],
    out_specs=(_row_spec, _at_spec),
    out_shape=_h_at_out)

_tc_mid = pl.pallas_call(
    _tc_mid_body, grid=(NBLK,),
    in_specs=[_row_spec, _row_spec, _b_spec, _full_spec, _attp_spec],
    out_specs=(_row_spec, _at_spec),
    out_shape=_h_at_out)

_tc_last = pl.pallas_call(
    _tc_last_body, grid=(NBLK,),
    in_specs=[_row_spec, _row_spec, _b_spec, _full_spec, _b_spec],
    out_specs=(_row_spec, _row_spec, pl.BlockSpec((8, ROWB), lambda i: (0, i))),
    out_shape=(jax.ShapeDtypeStruct((NPAD, D), jnp.float32),
               jax.ShapeDtypeStruct((NPAD, D), jnp.float32),
               jax.ShapeDtypeStruct((8, NPAD), jnp.int32)))


_mesh = plsc.VectorSubcoreMesh(core_axis_name="c", subcore_axis_name="s")


def _sc_alpha_body(at_hbm, src_hbm, dst_hbm, ex_hbm, den_hbm,
                   ai_v, aj_v, den_v, idi_v, src_v, dst_v, ex_v, den_sh, sem):
    cid = lax.axis_index("c")
    sid = lax.axis_index("s")
    wid = sid * 2 + cid
    base = wid * EW
    pltpu.async_copy(at_hbm.at[0], ai_v, sem)
    pltpu.async_copy(at_hbm.at[1], aj_v, sem)
    pltpu.async_copy(src_hbm.at[pl.ds(base, EW)], src_v, sem)
    pltpu.async_copy(dst_hbm.at[pl.ds(base, EW)], dst_v, sem)
    pltpu.make_async_copy(at_hbm.at[0], ai_v, sem).wait()
    pltpu.make_async_copy(at_hbm.at[1], aj_v, sem).wait()
    pltpu.make_async_copy(src_hbm.at[pl.ds(base, EW)], src_v, sem).wait()
    pltpu.make_async_copy(dst_hbm.at[pl.ds(base, EW)], dst_v, sem).wait()

    def mx(k, acc):
        return jnp.maximum(acc, ai_v[pl.ds(k * 16, 16)])
    acc = lax.fori_loop(0, NPAD // 16, mx, ai_v[pl.ds(0, 16)])
    lanes = lax.iota(jnp.int32, 16)
    for sh in (8, 4, 2, 1):
        ex_v[pl.ds(0, 16)] = acc
        acc = jnp.maximum(acc, plsc.load_gather(ex_v, [lanes ^ sh]))
    amax = acc

    for m in range(5):
        idi_v[0, pl.ds(m * 16, 16)] = lanes + m * 16

    zero16 = jnp.zeros((16,), jnp.float32)

    def zz(k, _):
        den_v[k // 8, pl.ds((k % 8) * 16, 16)] = zero16
        return 0
    lax.fori_loop(0, NPAD // 16, zz, 0)

    @pl.when(sid == 0)
    def _():
        pltpu.sync_copy(den_v, den_sh)
    plsc.subcore_barrier()

    def step(k, _):
        s16 = src_v[pl.ds(k * 16, 16)]
        d16 = dst_v[pl.ds(k * 16, 16)]
        ai = plsc.load_gather(ai_v, [d16])
        aj = plsc.load_gather(aj_v, [s16])
        ssum = ai + aj
        z = jnp.maximum(ssum, 0.2 * ssum)
        t = amax + aj
        c = jnp.maximum(t, 0.2 * t)
        ex = jnp.exp(z - c)
        ex_v[pl.ds(k * 16, 16)] = ex
        plsc.addupdate_scatter(
            den_v, [lax.shift_right_logical(s16, 7), jnp.bitwise_and(s16, 127)], ex)
        return 0
    lax.fori_loop(0, EW // 16, step, 0)

    pltpu.sync_copy(ex_v, ex_hbm.at[pl.ds(base, EW)])
    pltpu.sync_copy(den_v, den_sh.at[idi_v.at[0]], add=True)
    plsc.subcore_barrier()

    @pl.when(sid < 10)
    def _():
        pltpu.sync_copy(den_sh.at[pl.ds(sid * 8, 8)],
                        den_hbm.at[cid, pl.ds(sid * 8, 8)])


_sc_alpha = pl.kernel(
    _sc_alpha_body,
    out_type=(jax.ShapeDtypeStruct((EPAD,), jnp.float32),
              jax.ShapeDtypeStruct((2, NPAD // 128, 128), jnp.float32)),
    mesh=_mesh,
    compiler_params=pltpu.CompilerParams(needs_layout_passes=False),
    scratch_types=[
        pltpu.VMEM((NPAD,), jnp.float32),
        pltpu.VMEM((NPAD,), jnp.float32),
        pltpu.VMEM((NPAD // 128, 128), jnp.float32),
        pltpu.VMEM((1, NPAD // 128), jnp.int32),
        pltpu.VMEM((EW,), jnp.int32),
        pltpu.VMEM((EW,), jnp.int32),
        pltpu.VMEM((EW,), jnp.float32),
        pltpu.VMEM_SHARED((NPAD // 128, 128), jnp.float32),
        pltpu.SemaphoreType.DMA,
    ])


def _sc_agg_body(h_hbm, ex_hbm, dst_hbm, src_hbm, den_hbm, out_hbm,
                 den_v, src2, db0, db1, db2, db3, xb0, xb1, xb2, xb3,
                 dx0, dx1, dx2, wbuf,
                 rows0, rows1, rows2, srows0, srows1, srows2, out_sh,
                 g0, g1, g2, s0, s1, s2, d0, d1, d2, d3, x0, x1, x2, x3):
    cid = lax.axis_index("c")
    sid = lax.axis_index("s")
    wid = sid * 2 + cid
    rows = (rows0, rows1, rows2)
    srows = (srows0, srows1, srows2)
    db = (db0, db1, db2, db3)
    xb = (xb0, xb1, xb2, xb3)
    dx = (dx0, dx1, dx2)
    gsem = (g0, g1, g2)
    ssem = (s0, s1, s2)
    dsem = (d0, d1, d2, d3)
    xsem = (x0, x1, x2, x3)

    pltpu.sync_copy(src_hbm.at[wid], src2)

    def src_at(c):
        return src2.at[lax.shift_right_logical(c, 2),
                       pl.ds(jnp.bitwise_and(c, 3) * G, G)]

    pltpu.sync_copy(den_hbm.at[0], den_v)
    for buf, off, nn in ((rows0, 0, 32), (rows1, 32, 32), (rows2, 64, 16)):
        pltpu.sync_copy(den_hbm.at[1, pl.ds(off, nn)], buf.at[pl.ds(0, nn)])

        def rcp(k, _, buf=buf, off=off):
            r, i = k // 8, pl.ds((k % 8) * 16, 16)
            den_v[off + r, i] = 1.0 / (den_v[off + r, i] + buf[r, i] + 1e-16)
            return 0
        lax.fori_loop(0, nn * 8, rcp, 0)

    zero16 = jnp.zeros((16,), jnp.float32)

    def zrow(k, _):
        srows0[k // 8, pl.ds((k % 8) * 16, 16)] = zero16
        return 0
    lax.fori_loop(0, G * 8, zrow, 0)
    nz = NPAD // 16 // G
    for j in range(nz):
        pltpu.async_copy(srows0, out_sh.at[pl.ds(sid * (NPAD // 16) + j * G, G)],
                         g0)
    for j in range(nz):
        pltpu.make_async_copy(
            srows0, out_sh.at[pl.ds(sid * (NPAD // 16) + j * G, G)], g0).wait()
    plsc.subcore_barrier()

    for m in range(4):
        pltpu.async_copy(dst_hbm.at[wid, m], db[m], dsem[m])
        pltpu.async_copy(ex_hbm.at[wid, m], xb[m], xsem[m])
    pltpu.async_copy(h_hbm.at[src_at(0)], rows0, g0)
    pltpu.async_copy(h_hbm.at[src_at(1)], rows1, g1)
    pltpu.async_copy(h_hbm.at[src_at(2)], rows2, g2)

    def do_slot(j, c, k):
        p = k % 4
        q = k % 3
        pltpu.make_async_copy(dst_hbm.at[wid, c], db[p], dsem[p]).wait()
        pltpu.make_async_copy(ex_hbm.at[wid, c], xb[p], xsem[p]).wait()
        pltpu.make_async_copy(h_hbm.at[src_at(c)], rows[q], gsem[q]).wait()

        cr = lax.shift_right_logical(c, 2)
        cc0 = jnp.bitwise_and(c, 3) * G
        for m in range(G // 16):
            i = pl.ds(cc0 + m * 16, 16)
            s16 = src2[cr, i]
            wbuf[pl.ds(m * 16, 16)] = xb[p][pl.ds(m * 16, 16)] * plsc.load_gather(
                den_v, [lax.shift_right_logical(s16, 7),
                        jnp.bitwise_and(s16, 127)])

        @pl.when(c >= 3)
        def _():
            pltpu.make_async_copy(srows[q], out_sh.at[dx[q].at[0]],
                                  ssem[q]).wait()
        for m in range(G // 16):
            i = pl.ds(m * 16, 16)
            dx[q][0, i] = db[p][0, i]

        @plsc.parallel_loop(0, G, unroll=4)
        def scale(r):
            wb = plsc.load_gather(wbuf, [jnp.full((16,), r, jnp.int32)])
            for cc in range(8):
                i = pl.ds(cc * 16, 16)
                srows[q][r, i] = rows[q][r, i] * wb
        pltpu.async_copy(srows[q], out_sh.at[dx[q].at[0]], ssem[q], add=True)

        @pl.when(c + 4 < NCH)
        def _():
            pltpu.async_copy(dst_hbm.at[wid, c + 4], db[p], dsem[p])
            pltpu.async_copy(ex_hbm.at[wid, c + 4], xb[p], xsem[p])

        @pl.when(c + 3 < NCH)
        def _():
            pltpu.async_copy(h_hbm.at[src_at(c + 3)], rows[q], gsem[q])

    def dodeca(j, _):
        for k in range(12):
            do_slot(j, 12 * j + k, k)
        return 0
    lax.fori_loop(0, NCH // 12, dodeca, 0)
    for c in range(NCH - 3, NCH):
        q = c % 3
        pltpu.make_async_copy(srows[q], out_sh.at[dx[q].at[0]], ssem[q]).wait()
    plsc.subcore_barrier()

    for j in range(nz):
        r0 = sid * (NPAD // 16) + j * G
        pltpu.async_copy(out_sh.at[pl.ds(r0, G)], out_hbm.at[cid, pl.ds(r0, G)],
                         g0)
    for j in range(nz):
        r0 = sid * (NPAD // 16) + j * G
        pltpu.make_async_copy(out_sh.at[pl.ds(r0, G)],
                              out_hbm.at[cid, pl.ds(r0, G)], g0).wait()


_sc_agg = pl.kernel(
    _sc_agg_body,
    out_type=jax.ShapeDtypeStruct((2, NPAD, D), jnp.float32),
    mesh=_mesh,
    compiler_params=pltpu.CompilerParams(needs_layout_passes=False),
    scratch_types=[
        pltpu.VMEM((NPAD // 128, 128), jnp.float32),
        pltpu.VMEM((NCH // 4, 4 * G), jnp.int32),
        pltpu.VMEM((1, G), jnp.int32),
        pltpu.VMEM((1, G), jnp.int32),
        pltpu.VMEM((1, G), jnp.int32),
        pltpu.VMEM((1, G), jnp.int32),
        pltpu.VMEM((G,), jnp.float32),
        pltpu.VMEM((G,), jnp.float32),
        pltpu.VMEM((G,), jnp.float32),
        pltpu.VMEM((G,), jnp.float32),
        pltpu.VMEM((1, G), jnp.int32),
        pltpu.VMEM((1, G), jnp.int32),
        pltpu.VMEM((1, G), jnp.int32),
        pltpu.VMEM((G,), jnp.float32),
        pltpu.VMEM((G, D), jnp.float32),
        pltpu.VMEM((G, D), jnp.float32),
        pltpu.VMEM((G, D), jnp.float32),
        pltpu.VMEM((G, D), jnp.float32),
        pltpu.VMEM((G, D), jnp.float32),
        pltpu.VMEM((G, D), jnp.float32),
        pltpu.VMEM_SHARED((NPAD, D), jnp.float32),
        pltpu.SemaphoreType.DMA,
        pltpu.SemaphoreType.DMA,
        pltpu.SemaphoreType.DMA,
        pltpu.SemaphoreType.DMA,
        pltpu.SemaphoreType.DMA,
        pltpu.SemaphoreType.DMA,
        pltpu.SemaphoreType.DMA,
        pltpu.SemaphoreType.DMA,
        pltpu.SemaphoreType.DMA,
        pltpu.SemaphoreType.DMA,
        pltpu.SemaphoreType.DMA,
        pltpu.SemaphoreType.DMA,
        pltpu.SemaphoreType.DMA,
        pltpu.SemaphoreType.DMA,
    ])


def _attp(att):
    a = jnp.zeros((8, D), jnp.float32)
    return a.at[0].set(att[0, 0, :D]).at[1].set(att[0, 0, D:])


def kernel(x, edge_index, node_label, node_index,
           W1, att1, b1, W2, att2, b2, W3, att3, b3, outW, outb):
    del node_label
    x_pad = jnp.zeros((NPAD, D), jnp.float32).at[:N].set(x)
    loops = jnp.arange(N, dtype=jnp.int32)
    padi = jnp.full((EPAD - N_EDGES_TOTAL,), NPAD - 1, jnp.int32)
    src = jnp.concatenate([edge_index[0], loops, padi])
    dst = jnp.concatenate([edge_index[1], loops, padi])
    src3 = src.reshape(NW, NCH // 4, 4 * G)
    dst4 = dst.reshape(NW, NCH, 1, G)

    def layer(g_parts, W, att, first):
        attp = _attp(att)
        if first:
            h, aT = _tc_first(g_parts, W, attp)
        else:
            p, b_prev = g_parts
            h, aT = _tc_mid(p[0], p[1], b_prev.reshape(1, D), W, attp)
        ex, den = _sc_alpha(aT, src, dst)
        return _sc_agg(h, ex.reshape(NW, NCH, G), dst4, src3, den), h

    o1, _ = layer(x_pad, W1, att1, True)
    o2, _ = layer((o1, b1), W2, att2, False)
    o3, _ = layer((o2, b2), W3, att3, False)

    owp = jnp.zeros((D, D), jnp.float32).at[:3].set(outW)
    obp = jnp.zeros((1, D), jnp.float32).at[0, :3].set(outb)
    xe, lg, ypr = _tc_last(o3[0], o3[1], b3.reshape(1, D), owp, obp)

    x_embed = xe[:N]
    output = lg[:N, :3]
    ypred = ypr[0, :N]
    node_output = output[node_index]
    y_nodepred = ypred[node_index]
    return (x_embed, node_output, ypred, y_nodepred)
